# Initial kernel scaffold; baseline (speedup 1.0000x reference)
import functools

import jax
import jax.numpy as jnp
from jax import lax
from jax.experimental import pallas as pl
from jax.experimental.pallas import tpu as pltpu
from jax.experimental.pallas import tpu_sc as plsc

NC = 2
NS = 16
NW = NC * NS
L = 16
CH = 128


def _worker_id():
  return lax.axis_index("s") * NC + lax.axis_index("c")


def _iota():
  return lax.iota(jnp.int32, L)


def _gather_body(B, H, D1, uidx_hbm, eidx_hbm, pidx_hbm, dayv_hbm, h_hbm,
                 C_hbm, pemb_hbm, uemb_hbm, wr_hbm, wrb_hbm,
                 x_out, hp_out, cp_out, wr_out, wrb_out,
                 uidx_v, eidx_v, pidx_v, dayv_v, rows_v, xp_v, xu_v,
                 hp_v, cp_v, wrv_v, wrb_v, sem):
  ept = B // NW
  nch = ept // CH
  wid = _worker_id()
  base = wid * ept
  pltpu.sync_copy(uidx_hbm.at[pl.ds(base, ept)], uidx_v)
  pltpu.sync_copy(eidx_hbm.at[pl.ds(base, ept)], eidx_v)
  pltpu.sync_copy(pidx_hbm.at[pl.ds(base, ept)], pidx_v)
  pltpu.sync_copy(dayv_hbm, dayv_v)
  day = dayv_v[...]
  iot = _iota()

  def chunk(c, _):
    off = c * CH
    obase = base + off
    pltpu.async_copy(pemb_hbm.at[eidx_v.at[pl.ds(off, CH)]], xp_v, sem).wait()
    pltpu.async_copy(uemb_hbm.at[uidx_v.at[pl.ds(off, CH)]], xu_v, sem).wait()
    pltpu.async_copy(wr_hbm.at[pidx_v.at[pl.ds(off, CH)]], wrv_v, sem).wait()
    pltpu.async_copy(wrb_hbm.at[pidx_v.at[pl.ds(off, CH)]], wrb_v, sem).wait()

    def add_row(r, _):
      xp_v[r, pl.ds(0, L)] = xp_v[r, pl.ds(0, L)] + xu_v[r, pl.ds(0, L)]
      xp_v[r, pl.ds(L, L)] = xp_v[r, pl.ds(L, L)] + xu_v[r, pl.ds(L, L)]
      return 0

    lax.fori_loop(0, CH, add_row, 0)
    pltpu.sync_copy(xp_v, x_out.at[pl.ds(obase, CH)])
    pltpu.sync_copy(wrv_v, wr_out.at[pl.ds(obase, CH)])
    pltpu.sync_copy(wrb_v, wrb_out.at[pl.ds(obase, CH)])

    pltpu.async_copy(h_hbm.at[uidx_v.at[pl.ds(off, CH)]], rows_v, sem).wait()

    def ext_h(e, _):
      eb = jnp.full((L,), e, jnp.int32)
      hp_v[e, pl.ds(0, L)] = plsc.load_gather(rows_v, [eb, iot, day])
      hp_v[e, pl.ds(L, L)] = plsc.load_gather(rows_v, [eb, iot + L, day])
      return 0

    lax.fori_loop(0, CH, ext_h, 0)
    pltpu.sync_copy(hp_v, hp_out.at[pl.ds(obase, CH)])

    pltpu.async_copy(C_hbm.at[uidx_v.at[pl.ds(off, CH)]], rows_v, sem).wait()

    def ext_c(e, _):
      eb = jnp.full((L,), e, jnp.int32)
      cp_v[e, pl.ds(0, L)] = plsc.load_gather(rows_v, [eb, iot, day])
      cp_v[e, pl.ds(L, L)] = plsc.load_gather(rows_v, [eb, iot + L, day])
      return 0

    lax.fori_loop(0, CH, ext_c, 0)
    pltpu.sync_copy(cp_v, cp_out.at[pl.ds(obase, CH)])
    return 0

  lax.fori_loop(0, nch, chunk, 0)


def _scatter_body(B, U, H, D1, uidx_hbm, hn_hbm, cn_hbm, day1v_hbm,
                  h_ref, C_ref,
                  uall_v, w_v, wulist_v, welist_v, u128_v, ev128_v,
                  rows_v, hn_v, day1_v, sem):
  urange = (U + NW - 1) // NW
  wcap = w_v.shape[0]
  wid = _worker_id()
  lo = wid * urange
  hi = jnp.minimum(lo + urange, U)
  pltpu.sync_copy(uidx_hbm, uall_v)
  pltpu.sync_copy(day1v_hbm, day1_v)
  d1 = day1_v[...]
  iot = _iota()
  neg1 = jnp.full((L,), -1, jnp.int32)

  def winit(j, _):
    w_v[pl.ds(j * L, L)] = neg1
    return 0

  lax.fori_loop(0, wcap // L, winit, 0)

  def wpass(i, _):
    uv = uall_v[pl.ds(i * L, L)]
    mine = (uv >= lo) & (uv < hi)
    rel = jnp.where(mine, uv - lo, 0)
    plsc.store_scatter(w_v, [rel], iot + i * L, mask=mine)
    return 0

  lax.fori_loop(0, B // L, wpass, 0)

  def enum(j, m):
    wv = w_v[pl.ds(j * L, L)]
    has = wv >= 0
    plsc.store_compressed(wulist_v.at[pl.ds(m, L)], lo + iot + j * L,
                          mask=has)
    plsc.store_compressed(welist_v.at[pl.ds(m, L)], wv, mask=has)
    return m + jnp.max(plsc.all_reduce_population_count(has))

  m_tot = lax.fori_loop(0, wcap // L, enum, jnp.int32(0))

  @pl.when(m_tot > 0)
  def _():
    nch = (m_tot + CH - 1) // CH

    def chunk(c, _):
      for s in range(CH // L):
        pos = jnp.minimum(c * CH + s * L + iot, m_tot - 1)
        u128_v[pl.ds(s * L, L)] = plsc.load_gather(wulist_v, [pos])
        ev128_v[pl.ds(s * L, L)] = plsc.load_gather(welist_v, [pos])

      def insert(e, _):
        eb = jnp.full((L,), e, jnp.int32)
        plsc.store_scatter(rows_v, [eb, iot, d1], hn_v[e, pl.ds(0, L)])
        plsc.store_scatter(rows_v, [eb, iot + L, d1], hn_v[e, pl.ds(L, L)])
        return 0

      pltpu.async_copy(h_ref.at[u128_v], rows_v, sem).wait()
      pltpu.async_copy(hn_hbm.at[ev128_v], hn_v, sem).wait()
      lax.fori_loop(0, CH, insert, 0)
      pltpu.async_copy(rows_v, h_ref.at[u128_v], sem).wait()

      pltpu.async_copy(C_ref.at[u128_v], rows_v, sem).wait()
      pltpu.async_copy(cn_hbm.at[ev128_v], hn_v, sem).wait()
      lax.fori_loop(0, CH, insert, 0)
      pltpu.async_copy(rows_v, C_ref.at[u128_v], sem).wait()
      return 0

    lax.fori_loop(0, nch, chunk, 0)


def _lstm_body(x_ref, hp_ref, cp_ref, wr_ref, wrb_ref, wcat_ref, ucat_ref,
               bcat_ref, hn_ref, cn_ref, sc_ref):
  x = x_ref[...]
  hp = hp_ref[...]
  g = (jnp.dot(x, wcat_ref[...], preferred_element_type=jnp.float32)
       + jnp.dot(hp, ucat_ref[...], preferred_element_type=jnp.float32)
       + bcat_ref[...])
  h = x.shape[1]
  gi = jax.nn.sigmoid(g[:, :h])
  gf = jax.nn.sigmoid(g[:, h:2 * h])
  go = jax.nn.sigmoid(g[:, 2 * h:3 * h])
  gc = jnp.tanh(g[:, 3 * h:])
  cn = gf * cp_ref[...] + gi * gc
  hn = go * jnp.tanh(cn)
  hn_ref[...] = hn
  cn_ref[...] = cn
  s = jnp.sum(hn * wr_ref[...], axis=1)[None, :] + wrb_ref[...]
  sc_ref[...] = jax.nn.sigmoid(s)


def kernel(user_idx, emb_idx, problem_idx, day, h, C, user_emb, problem_emb,
           Wi, Wf, Wo, Wc, Ui, Uf, Uo, Uc, Ui_b, Uf_b, Uo_b, Uc_b, Wr, Wr_b):
  B = user_idx.shape[0]
  U, H, D1 = h.shape
  uidx = user_idx.astype(jnp.int32)
  eidx = emb_idx.astype(jnp.int32)
  pidx = problem_idx.astype(jnp.int32)
  dayv = jnp.full((L,), day, jnp.int32)
  mesh = plsc.VectorSubcoreMesh(core_axis_name="c", subcore_axis_name="s",
                                num_cores=NC, num_subcores=NS)

  f32 = jnp.float32
  i32 = jnp.int32
  ept = B // NW

  gather = pl.kernel(
      functools.partial(_gather_body, B, H, D1),
      out_type=(
          jax.ShapeDtypeStruct((B, H), f32),
          jax.ShapeDtypeStruct((B, H), f32),
          jax.ShapeDtypeStruct((B, H), f32),
          jax.ShapeDtypeStruct((B, H), f32),
          jax.ShapeDtypeStruct((B,), f32),
      ),
      mesh=mesh,
      scratch_types=[
          pltpu.VMEM((ept,), i32),
          pltpu.VMEM((ept,), i32),
          pltpu.VMEM((ept,), i32),
          pltpu.VMEM((L,), i32),
          pltpu.VMEM((CH, H, D1), f32),
          pltpu.VMEM((CH, H), f32),
          pltpu.VMEM((CH, H), f32),
          pltpu.VMEM((CH, H), f32),
          pltpu.VMEM((CH, H), f32),
          pltpu.VMEM((CH, H), f32),
          pltpu.VMEM((CH,), f32),
          pltpu.SemaphoreType.DMA,
      ],
      name="sc_lstm_gather",
  )
  x, hp, cp, wr, wrb = gather(uidx, eidx, pidx, dayv, h, C, problem_emb,
                              user_emb, Wr, Wr_b)

  wcat = jnp.concatenate([Wi.T, Wf.T, Wo.T, Wc.T], axis=1)
  ucat = jnp.concatenate([Ui.T, Uf.T, Uo.T, Uc.T], axis=1)
  bcat = jnp.concatenate([Ui_b, Uf_b, Uo_b, Uc_b])[None, :]

  blk = 512
  nblk = B // blk
  hn, cn, score2d = pl.pallas_call(
      _lstm_body,
      grid=(nblk,),
      in_specs=[
          pl.BlockSpec((blk, H), lambda i: (i, 0)),
          pl.BlockSpec((blk, H), lambda i: (i, 0)),
          pl.BlockSpec((blk, H), lambda i: (i, 0)),
          pl.BlockSpec((blk, H), lambda i: (i, 0)),
          pl.BlockSpec((1, blk), lambda i: (i, 0)),
          pl.BlockSpec((H, 4 * H), lambda i: (0, 0)),
          pl.BlockSpec((H, 4 * H), lambda i: (0, 0)),
          pl.BlockSpec((1, 4 * H), lambda i: (0, 0)),
      ],
      out_specs=[
          pl.BlockSpec((blk, H), lambda i: (i, 0)),
          pl.BlockSpec((blk, H), lambda i: (i, 0)),
          pl.BlockSpec((1, blk), lambda i: (i, 0)),
      ],
      out_shape=[
          jax.ShapeDtypeStruct((B, H), f32),
          jax.ShapeDtypeStruct((B, H), f32),
          jax.ShapeDtypeStruct((nblk, blk), f32),
      ],
      name="tc_lstm_gates",
  )(x, hp, cp, wr, wrb.reshape(nblk, blk), wcat, ucat, bcat)
  score = score2d.reshape(B)

  urange = (U + NW - 1) // NW
  wcap = ((urange + L) + L - 1) // L * L

  h_ref = jax.new_ref(h)
  C_ref = jax.new_ref(C)
  scatter = pl.kernel(
      functools.partial(_scatter_body, B, U, H, D1),
      out_type=(),
      mesh=mesh,
      scratch_types=[
          pltpu.VMEM((B,), i32),
          pltpu.VMEM((wcap,), i32),
          pltpu.VMEM((wcap,), i32),
          pltpu.VMEM((wcap,), i32),
          pltpu.VMEM((CH,), i32),
          pltpu.VMEM((CH,), i32),
          pltpu.VMEM((CH, H, D1), f32),
          pltpu.VMEM((CH, H), f32),
          pltpu.VMEM((L,), i32),
          pltpu.SemaphoreType.DMA,
      ],
      name="sc_lstm_scatter",
  )
  scatter(uidx, hn, cn, dayv + 1, h_ref, C_ref)
  return jax.freeze(h_ref), jax.freeze(C_ref), score


# speedup vs baseline: 1.8602x; 1.8602x over previous
import functools

import jax
import jax.numpy as jnp
from jax import lax
from jax.experimental import pallas as pl
from jax.experimental.pallas import tpu as pltpu
from jax.experimental.pallas import tpu_sc as plsc

NC = 2
NS = 16
NW = NC * NS
L = 16
CH = 128


def _worker_id():
  return lax.axis_index("s") * NC + lax.axis_index("c")


def _iota():
  return lax.iota(jnp.int32, L)


def _gather_body(B, H, D1, uidx_hbm, eidx_hbm, pidx_hbm, dayv_hbm, h_hbm,
                 C_hbm, pemb_hbm, uemb_hbm, wr_hbm, wrb_hbm,
                 x_out, hp_out, cp_out, wr_out, wrb_out,
                 uidx_v, eidx_v, pidx_v, dayv_v, rows_v, xp_v, xu_v,
                 hp_v, cp_v, wrv_v, wrb_v, sem):
  ept = B // NW
  nch = ept // CH
  wid = _worker_id()
  base = wid * ept
  pltpu.sync_copy(uidx_hbm.at[pl.ds(base, ept)], uidx_v)
  pltpu.sync_copy(eidx_hbm.at[pl.ds(base, ept)], eidx_v)
  pltpu.sync_copy(pidx_hbm.at[pl.ds(base, ept)], pidx_v)
  pltpu.sync_copy(dayv_hbm, dayv_v)
  day = dayv_v[...]
  iot = _iota()

  def chunk(c, _):
    off = c * CH
    obase = base + off
    pltpu.async_copy(pemb_hbm.at[eidx_v.at[pl.ds(off, CH)]], xp_v, sem).wait()
    pltpu.async_copy(uemb_hbm.at[uidx_v.at[pl.ds(off, CH)]], xu_v, sem).wait()
    pltpu.async_copy(wr_hbm.at[pidx_v.at[pl.ds(off, CH)]], wrv_v, sem).wait()
    pltpu.async_copy(wrb_hbm.at[pidx_v.at[pl.ds(off, CH)]], wrb_v, sem).wait()

    def add_row(r, _):
      xp_v[r, pl.ds(0, L)] = xp_v[r, pl.ds(0, L)] + xu_v[r, pl.ds(0, L)]
      xp_v[r, pl.ds(L, L)] = xp_v[r, pl.ds(L, L)] + xu_v[r, pl.ds(L, L)]
      return 0

    lax.fori_loop(0, CH, add_row, 0)
    pltpu.sync_copy(xp_v, x_out.at[pl.ds(obase, CH)])
    pltpu.sync_copy(wrv_v, wr_out.at[pl.ds(obase, CH)])
    pltpu.sync_copy(wrb_v, wrb_out.at[pl.ds(obase, CH)])

    j0 = iot * D1 + day
    j1 = j0 + L * D1
    pltpu.async_copy(h_hbm.at[uidx_v.at[pl.ds(off, CH)]], rows_v, sem).wait()

    def ext_h(e, _):
      eb = jnp.full((L,), e, jnp.int32)
      hp_v[e, pl.ds(0, L)] = plsc.load_gather(rows_v, [eb, j0])
      hp_v[e, pl.ds(L, L)] = plsc.load_gather(rows_v, [eb, j1])
      return 0

    lax.fori_loop(0, CH, ext_h, 0)
    pltpu.sync_copy(hp_v, hp_out.at[pl.ds(obase, CH)])

    pltpu.async_copy(C_hbm.at[uidx_v.at[pl.ds(off, CH)]], rows_v, sem).wait()

    def ext_c(e, _):
      eb = jnp.full((L,), e, jnp.int32)
      cp_v[e, pl.ds(0, L)] = plsc.load_gather(rows_v, [eb, j0])
      cp_v[e, pl.ds(L, L)] = plsc.load_gather(rows_v, [eb, j1])
      return 0

    lax.fori_loop(0, CH, ext_c, 0)
    pltpu.sync_copy(cp_v, cp_out.at[pl.ds(obase, CH)])
    return 0

  lax.fori_loop(0, nch, chunk, 0)


def _scatter_body(B, U, H, D1, uidx_hbm, hn_hbm, cn_hbm, day1v_hbm,
                  h_ref, C_ref,
                  uall_v, w_v, wulist_v, welist_v, u128_v, ev128_v,
                  rows_v, hn_v, day1_v, sem):
  urange = (U + NW - 1) // NW
  wcap = w_v.shape[0]
  wid = _worker_id()
  lo = wid * urange
  hi = jnp.minimum(lo + urange, U)
  pltpu.sync_copy(uidx_hbm, uall_v)
  pltpu.sync_copy(day1v_hbm, day1_v)
  d1 = day1_v[...]
  iot = _iota()
  neg1 = jnp.full((L,), -1, jnp.int32)

  def winit(j, _):
    w_v[pl.ds(j * L, L)] = neg1
    return 0

  lax.fori_loop(0, wcap // L, winit, 0)

  def wpass(i, _):
    uv = uall_v[pl.ds(i * L, L)]
    mine = (uv >= lo) & (uv < hi)
    rel = jnp.where(mine, uv - lo, 0)
    plsc.store_scatter(w_v, [rel], iot + i * L, mask=mine)
    return 0

  lax.fori_loop(0, B // L, wpass, 0)

  def enum(j, m):
    wv = w_v[pl.ds(j * L, L)]
    has = wv >= 0
    plsc.store_compressed(wulist_v.at[pl.ds(m, L)], lo + iot + j * L,
                          mask=has)
    plsc.store_compressed(welist_v.at[pl.ds(m, L)], wv, mask=has)
    return m + jnp.max(plsc.all_reduce_population_count(has))

  m_tot = lax.fori_loop(0, wcap // L, enum, jnp.int32(0))

  @pl.when(m_tot > 0)
  def _():
    nch = (m_tot + CH - 1) // CH
    j0 = iot * D1 + d1
    j1 = j0 + L * D1

    def chunk(c, _):
      for s in range(CH // L):
        pos = jnp.minimum(c * CH + s * L + iot, m_tot - 1)
        u128_v[pl.ds(s * L, L)] = plsc.load_gather(wulist_v, [pos])
        ev128_v[pl.ds(s * L, L)] = plsc.load_gather(welist_v, [pos])

      def insert(e, _):
        eb = jnp.full((L,), e, jnp.int32)
        plsc.store_scatter(rows_v, [eb, j0], hn_v[e, pl.ds(0, L)])
        plsc.store_scatter(rows_v, [eb, j1], hn_v[e, pl.ds(L, L)])
        return 0

      pltpu.async_copy(h_ref.at[u128_v], rows_v, sem).wait()
      pltpu.async_copy(hn_hbm.at[ev128_v], hn_v, sem).wait()
      lax.fori_loop(0, CH, insert, 0)
      pltpu.async_copy(rows_v, h_ref.at[u128_v], sem).wait()

      pltpu.async_copy(C_ref.at[u128_v], rows_v, sem).wait()
      pltpu.async_copy(cn_hbm.at[ev128_v], hn_v, sem).wait()
      lax.fori_loop(0, CH, insert, 0)
      pltpu.async_copy(rows_v, C_ref.at[u128_v], sem).wait()
      return 0

    lax.fori_loop(0, nch, chunk, 0)


def _lstm_body(x_ref, hp_ref, cp_ref, wr_ref, wrb_ref, wcat_ref, ucat_ref,
               bcat_ref, hn_ref, cn_ref, sc_ref):
  x = x_ref[...]
  hp = hp_ref[...]
  g = (jnp.dot(x, wcat_ref[...], preferred_element_type=jnp.float32)
       + jnp.dot(hp, ucat_ref[...], preferred_element_type=jnp.float32)
       + bcat_ref[...])
  h = x.shape[1]
  gi = jax.nn.sigmoid(g[:, :h])
  gf = jax.nn.sigmoid(g[:, h:2 * h])
  go = jax.nn.sigmoid(g[:, 2 * h:3 * h])
  gc = jnp.tanh(g[:, 3 * h:])
  cn = gf * cp_ref[...] + gi * gc
  hn = go * jnp.tanh(cn)
  hn_ref[...] = hn
  cn_ref[...] = cn
  s = jnp.sum(hn * wr_ref[...], axis=1)[None, None, :] + wrb_ref[...]
  sc_ref[...] = jax.nn.sigmoid(s)


def kernel(user_idx, emb_idx, problem_idx, day, h, C, user_emb, problem_emb,
           Wi, Wf, Wo, Wc, Ui, Uf, Uo, Uc, Ui_b, Uf_b, Uo_b, Uc_b, Wr, Wr_b):
  B = user_idx.shape[0]
  U, H, D1 = h.shape
  uidx = user_idx.astype(jnp.int32)
  eidx = emb_idx.astype(jnp.int32)
  pidx = problem_idx.astype(jnp.int32)
  dayv = jnp.full((L,), day, jnp.int32)
  mesh = plsc.VectorSubcoreMesh(core_axis_name="c", subcore_axis_name="s",
                                num_cores=NC, num_subcores=NS)
  sc_params = pltpu.CompilerParams(needs_layout_passes=False,
                                   use_tc_tiling_on_sc=False)

  f32 = jnp.float32
  i32 = jnp.int32
  ept = B // NW

  gather = pl.kernel(
      functools.partial(_gather_body, B, H, D1),
      out_type=(
          jax.ShapeDtypeStruct((B, H), f32),
          jax.ShapeDtypeStruct((B, H), f32),
          jax.ShapeDtypeStruct((B, H), f32),
          jax.ShapeDtypeStruct((B, H), f32),
          jax.ShapeDtypeStruct((B,), f32),
      ),
      mesh=mesh,
      scratch_types=[
          pltpu.VMEM((ept,), i32),
          pltpu.VMEM((ept,), i32),
          pltpu.VMEM((ept,), i32),
          pltpu.VMEM((L,), i32),
          pltpu.VMEM((CH, H * D1), f32),
          pltpu.VMEM((CH, H), f32),
          pltpu.VMEM((CH, H), f32),
          pltpu.VMEM((CH, H), f32),
          pltpu.VMEM((CH, H), f32),
          pltpu.VMEM((CH, H), f32),
          pltpu.VMEM((CH,), f32),
          pltpu.SemaphoreType.DMA,
      ],
      compiler_params=sc_params,
      name="sc_lstm_gather",
  )
  h2 = h.reshape(U, H * D1)
  C2 = C.reshape(U, H * D1)
  x, hp, cp, wr, wrb = gather(uidx, eidx, pidx, dayv, h2, C2, problem_emb,
                              user_emb, Wr, Wr_b)

  wcat = jnp.concatenate([Wi.T, Wf.T, Wo.T, Wc.T], axis=1)
  ucat = jnp.concatenate([Ui.T, Uf.T, Uo.T, Uc.T], axis=1)
  bcat = jnp.concatenate([Ui_b, Uf_b, Uo_b, Uc_b])[None, :]

  blk = 512
  nblk = B // blk
  hn, cn, score2d = pl.pallas_call(
      _lstm_body,
      grid=(nblk,),
      in_specs=[
          pl.BlockSpec((blk, H), lambda i: (i, 0)),
          pl.BlockSpec((blk, H), lambda i: (i, 0)),
          pl.BlockSpec((blk, H), lambda i: (i, 0)),
          pl.BlockSpec((blk, H), lambda i: (i, 0)),
          pl.BlockSpec((1, 1, blk), lambda i: (i, 0, 0)),
          pl.BlockSpec((H, 4 * H), lambda i: (0, 0)),
          pl.BlockSpec((H, 4 * H), lambda i: (0, 0)),
          pl.BlockSpec((1, 4 * H), lambda i: (0, 0)),
      ],
      out_specs=[
          pl.BlockSpec((blk, H), lambda i: (i, 0)),
          pl.BlockSpec((blk, H), lambda i: (i, 0)),
          pl.BlockSpec((1, 1, blk), lambda i: (i, 0, 0)),
      ],
      out_shape=[
          jax.ShapeDtypeStruct((B, H), f32),
          jax.ShapeDtypeStruct((B, H), f32),
          jax.ShapeDtypeStruct((nblk, 1, blk), f32),
      ],
      name="tc_lstm_gates",
  )(x, hp, cp, wr, wrb.reshape(nblk, 1, blk), wcat, ucat, bcat)
  score = score2d.reshape(B)

  urange = (U + NW - 1) // NW
  wcap = ((urange + L) + L - 1) // L * L

  h_ref = jax.new_ref(h2)
  C_ref = jax.new_ref(C2)
  scatter = pl.kernel(
      functools.partial(_scatter_body, B, U, H, D1),
      out_type=(),
      mesh=mesh,
      scratch_types=[
          pltpu.VMEM((B,), i32),
          pltpu.VMEM((wcap,), i32),
          pltpu.VMEM((wcap,), i32),
          pltpu.VMEM((wcap,), i32),
          pltpu.VMEM((CH,), i32),
          pltpu.VMEM((CH,), i32),
          pltpu.VMEM((CH, H * D1), f32),
          pltpu.VMEM((CH, H), f32),
          pltpu.VMEM((L,), i32),
          pltpu.SemaphoreType.DMA,
      ],
      compiler_params=sc_params,
      name="sc_lstm_scatter",
  )
  scatter(uidx, hn, cn, dayv + 1, h_ref, C_ref)
  h_out = jax.freeze(h_ref).reshape(U, H, D1)
  C_out = jax.freeze(C_ref).reshape(U, H, D1)
  return h_out, C_out, score


# speedup vs baseline: 7.5518x; 4.0598x over previous
import functools

import jax
import jax.numpy as jnp
from jax import lax
from jax.experimental import pallas as pl
from jax.experimental.pallas import tpu as pltpu
from jax.experimental.pallas import tpu_sc as plsc

NC = 2
NS = 16
NW = NC * NS
L = 16
CH = 128
H_LANES = 32


def _worker_id():
  return lax.axis_index("s") * NC + lax.axis_index("c")


def _iota():
  return lax.iota(jnp.int32, L)


def _gather_body(B, H, uidx_hbm, eidx_hbm, pidx_hbm, pemb_hbm, uemb_hbm,
                 wr_hbm, wrb_hbm, x_out, wr_out, wrb_out,
                 uidx_v, eidx_v, pidx_v, xp_v, xu_v, wrv_v, wrb_v, sem):
  ept = B // NW
  nch = ept // CH
  wid = _worker_id()
  base = wid * ept
  pltpu.sync_copy(uidx_hbm.at[pl.ds(base, ept)], uidx_v)
  pltpu.sync_copy(eidx_hbm.at[pl.ds(base, ept)], eidx_v)
  pltpu.sync_copy(pidx_hbm.at[pl.ds(base, ept)], pidx_v)

  def chunk(c, _):
    off = c * CH
    obase = base + off
    pltpu.async_copy(pemb_hbm.at[eidx_v.at[pl.ds(off, CH)]], xp_v, sem).wait()
    pltpu.async_copy(uemb_hbm.at[uidx_v.at[pl.ds(off, CH)]], xu_v, sem).wait()
    pltpu.async_copy(wr_hbm.at[pidx_v.at[pl.ds(off, CH)]], wrv_v, sem).wait()
    pltpu.async_copy(wrb_hbm.at[pidx_v.at[pl.ds(off, CH)]], wrb_v, sem).wait()

    def add_row(r, _):
      xp_v[r, pl.ds(0, L)] = xp_v[r, pl.ds(0, L)] + xu_v[r, pl.ds(0, L)]
      xp_v[r, pl.ds(L, L)] = xp_v[r, pl.ds(L, L)] + xu_v[r, pl.ds(L, L)]
      return 0

    lax.fori_loop(0, CH, add_row, 0)
    pltpu.sync_copy(xp_v, x_out.at[pl.ds(obase, CH)])
    pltpu.sync_copy(wrv_v, wr_out.at[pl.ds(obase, CH)])
    pltpu.sync_copy(wrb_v, wrb_out.at[pl.ds(obase, CH)])
    return 0

  lax.fori_loop(0, nch, chunk, 0)


def _state_gather_body(B, U, uidx_hbm, dayv_hbm, h_hbm, C_hbm, hpcp_out,
                       seg_v, evlist_v, uvlist_v, u128_v, ev128_v,
                       winh_v, winc_v, st_v, dayv_v, sem):
  tr, last = _tile_range(U)
  seg = seg_v.shape[0]
  nseg = B // seg
  wid = _worker_id()
  lo = wid * tr
  pltpu.sync_copy(dayv_hbm, dayv_v)
  ds_ = jnp.max(dayv_v[...])
  iot = _iota()

  def do_windows(width, nsw):
    for sw in range(nsw):
      slo = lo + sw * width
      shi = jnp.minimum(slo + width, U)
      wdst_h = winh_v.at[:, pl.ds(0, width)]
      wdst_c = winc_v.at[:, pl.ds(0, width)]
      pltpu.async_copy(h_hbm.at[ds_, :, pl.ds(slo, width)], wdst_h, sem).wait()
      pltpu.async_copy(C_hbm.at[ds_, :, pl.ds(slo, width)], wdst_c, sem).wait()
      for sg in range(nseg):
        pltpu.sync_copy(uidx_hbm.at[pl.ds(sg * seg, seg)], seg_v)

        def scan(i, m):
          uv = seg_v[pl.ds(i * L, L)]
          mine = (uv >= slo) & (uv < shi)
          plsc.store_compressed(evlist_v.at[pl.ds(m, L)],
                                sg * seg + i * L + iot, mask=mine)
          plsc.store_compressed(uvlist_v.at[pl.ds(m, L)], uv - slo,
                                mask=mine)
          return m + jnp.max(plsc.all_reduce_population_count(mine))

        m_tot = lax.fori_loop(0, seg // L, scan, jnp.int32(0))

        @pl.when(m_tot > 0)
        def _():
          nch = (m_tot + CH - 1) // CH

          def chunk(c, _):
            for s in range(CH // L):
              pos = jnp.minimum(c * CH + s * L + iot, m_tot - 1)
              u128_v[pl.ds(s * L, L)] = plsc.load_gather(uvlist_v, [pos])
              ev128_v[pl.ds(s * L, L)] = plsc.load_gather(evlist_v, [pos])

            def ext(e, _):
              ub = plsc.load_gather(u128_v, [jnp.full((L,), e, jnp.int32)])
              st_v[e, pl.ds(0, L)] = plsc.load_gather(winh_v, [iot, ub])
              st_v[e, pl.ds(L, L)] = plsc.load_gather(winh_v, [iot + L, ub])
              st_v[e, pl.ds(2 * L, L)] = plsc.load_gather(winc_v, [iot, ub])
              st_v[e, pl.ds(3 * L, L)] = plsc.load_gather(winc_v,
                                                          [iot + L, ub])
              return 0

            lax.fori_loop(0, CH, ext, 0)
            pltpu.async_copy(st_v, hpcp_out.at[ev128_v], sem).wait()
            return 0

          lax.fori_loop(0, nch, chunk, 0)

  last_pad = (last + 127) // 128 * 128

  @pl.when(wid < NW - 1)
  def _():
    do_windows(tr // 5, 5)

  @pl.when(wid == NW - 1)
  def _():
    do_windows(last_pad, 1)


def _tile_range(U):
  tr = (((U + NW - 1) // NW) + 127) // 128 * 128
  last = U - (NW - 1) * tr
  assert 0 < last <= tr
  return tr, last


def _scatter_body(B, U, uidx_hbm, hncn_hbm, day1v_hbm, h_ref, C_ref,
                  seg_v, w_v, wulist_v, welist_v, u128_v, ev128_v,
                  win_v, hv_v, day1_v, sem):
  tr, last = _tile_range(U)
  seg = seg_v.shape[0]
  nseg = B // seg
  wid = _worker_id()
  lo = wid * tr
  hi = jnp.minimum(lo + tr, U)
  pltpu.sync_copy(day1v_hbm, day1_v)
  d1s = jnp.max(day1_v[...])
  iot = _iota()
  neg1 = jnp.full((L,), -1, jnp.int32)

  def winit(j, _):
    w_v[pl.ds(j * L, L)] = neg1
    return 0

  lax.fori_loop(0, tr // L, winit, 0)

  for sg in range(nseg):
    pltpu.sync_copy(uidx_hbm.at[pl.ds(sg * seg, seg)], seg_v)

    def wpass(i, _):
      uv = seg_v[pl.ds(i * L, L)]
      mine = (uv >= lo) & (uv < hi)
      rel = jnp.where(mine, uv - lo, 0)
      plsc.store_scatter(w_v, [rel], sg * seg + i * L + iot, mask=mine)
      return 0

    lax.fori_loop(0, seg // L, wpass, 0)

  def do_windows(width, nsw):
    for sw in range(nsw):
      swbase = sw * width
      slo = lo + swbase

      def enum(j, m):
        wv = w_v[pl.ds(swbase + j * L, L)]
        has = wv >= 0
        plsc.store_compressed(wulist_v.at[pl.ds(m, L)], j * L + iot,
                              mask=has)
        plsc.store_compressed(welist_v.at[pl.ds(m, L)], wv, mask=has)
        return m + jnp.max(plsc.all_reduce_population_count(has))

      m_tot = lax.fori_loop(0, width // L, enum, jnp.int32(0))

      for ref, lane0 in ((h_ref, 0), (C_ref, H_LANES)):
        dst = win_v.at[:, pl.ds(0, width)]
        pltpu.async_copy(ref.at[d1s, :, pl.ds(slo, width)], dst, sem).wait()

        @pl.when(m_tot > 0)
        def _():
          nch = (m_tot + CH - 1) // CH

          def chunk(c, _):
            for s in range(CH // L):
              pos = jnp.minimum(c * CH + s * L + iot, m_tot - 1)
              u128_v[pl.ds(s * L, L)] = plsc.load_gather(wulist_v, [pos])
              ev128_v[pl.ds(s * L, L)] = plsc.load_gather(welist_v, [pos])
            pltpu.async_copy(hncn_hbm.at[ev128_v], hv_v, sem).wait()

            def ins(e, _):
              ub = plsc.load_gather(u128_v, [jnp.full((L,), e, jnp.int32)])
              plsc.store_scatter(win_v, [iot, ub], hv_v[e, pl.ds(lane0, L)])
              plsc.store_scatter(win_v, [iot + L, ub],
                                 hv_v[e, pl.ds(lane0 + L, L)])
              return 0

            lax.fori_loop(0, CH, ins, 0)
            return 0

          lax.fori_loop(0, nch, chunk, 0)

        pltpu.async_copy(dst, ref.at[d1s, :, pl.ds(slo, width)], sem).wait()

  last_pad = (last + 127) // 128 * 128

  @pl.when(wid < NW - 1)
  def _():
    do_windows(tr // 5, 5)

  @pl.when(wid == NW - 1)
  def _():
    do_windows(last_pad, 1)


def _lstm_body(x_ref, hpcp_ref, wr_ref, wrb_ref, wcat_ref, ucat_ref,
               bcat_ref, hncn_ref, sc_ref):
  x = x_ref[...]
  h = x.shape[1]
  hp = hpcp_ref[:, :h]
  cp = hpcp_ref[:, h:2 * h]
  g = (jnp.dot(x, wcat_ref[...], preferred_element_type=jnp.float32)
       + jnp.dot(hp, ucat_ref[...], preferred_element_type=jnp.float32)
       + bcat_ref[...])
  gi = jax.nn.sigmoid(g[:, :h])
  gf = jax.nn.sigmoid(g[:, h:2 * h])
  go = jax.nn.sigmoid(g[:, 2 * h:3 * h])
  gc = jnp.tanh(g[:, 3 * h:])
  cn = gf * cp + gi * gc
  hn = go * jnp.tanh(cn)
  pad = jnp.zeros((x.shape[0], 2 * h), jnp.float32)
  hncn_ref[...] = jnp.concatenate([hn, cn, pad], axis=1)
  s = jnp.sum(hn * wr_ref[...], axis=1)[None, None, :] + wrb_ref[...]
  sc_ref[...] = jax.nn.sigmoid(s)


def kernel(user_idx, emb_idx, problem_idx, day, h, C, user_emb, problem_emb,
           Wi, Wf, Wo, Wc, Ui, Uf, Uo, Uc, Ui_b, Uf_b, Uo_b, Uc_b, Wr, Wr_b):
  B = user_idx.shape[0]
  U, H, D1 = h.shape
  uidx = user_idx.astype(jnp.int32)
  eidx = emb_idx.astype(jnp.int32)
  pidx = problem_idx.astype(jnp.int32)
  dayv = jnp.full((L,), day, jnp.int32)
  mesh = plsc.VectorSubcoreMesh(core_axis_name="c", subcore_axis_name="s",
                                num_cores=NC, num_subcores=NS)
  sc_params = pltpu.CompilerParams(needs_layout_passes=False,
                                   use_tc_tiling_on_sc=False)

  f32 = jnp.float32
  i32 = jnp.int32
  ept = B // NW

  gather = pl.kernel(
      functools.partial(_gather_body, B, H),
      out_type=(
          jax.ShapeDtypeStruct((B, H), f32),
          jax.ShapeDtypeStruct((B, H), f32),
          jax.ShapeDtypeStruct((B,), f32),
      ),
      mesh=mesh,
      scratch_types=[
          pltpu.VMEM((ept,), i32),
          pltpu.VMEM((ept,), i32),
          pltpu.VMEM((ept,), i32),
          pltpu.VMEM((CH, H), f32),
          pltpu.VMEM((CH, H), f32),
          pltpu.VMEM((CH, H), f32),
          pltpu.VMEM((CH,), f32),
          pltpu.SemaphoreType.DMA,
      ],
      compiler_params=sc_params,
      name="sc_lstm_gather",
  )
  x, wr, wrb = gather(uidx, eidx, pidx, problem_emb, user_emb, Wr, Wr_b)

  tr, _last = _tile_range(U)
  seg = 4096
  wmax = max(tr // 5, (_last + 127) // 128 * 128)
  hT = jnp.transpose(h, (2, 1, 0))
  CT = jnp.transpose(C, (2, 1, 0))
  sc_tiled_params = pltpu.CompilerParams(needs_layout_passes=False,
                                         use_tc_tiling_on_sc=True,
                                         disable_bounds_checks=True)
  state_gather = pl.kernel(
      functools.partial(_state_gather_body, B, U),
      out_type=jax.ShapeDtypeStruct((B, 4 * H), f32),
      mesh=mesh,
      scratch_types=[
          pltpu.VMEM((seg,), i32),
          pltpu.VMEM((seg,), i32),
          pltpu.VMEM((seg,), i32),
          pltpu.VMEM((CH,), i32),
          pltpu.VMEM((CH,), i32),
          pltpu.VMEM((H, wmax), f32),
          pltpu.VMEM((H, wmax), f32),
          pltpu.VMEM((CH, 4 * H), f32),
          pltpu.VMEM((L,), i32),
          pltpu.SemaphoreType.DMA,
      ],
      compiler_params=sc_tiled_params,
      name="sc_lstm_state_gather",
  )
  hpcp = state_gather(uidx, dayv, hT, CT)

  wcat = jnp.concatenate([Wi.T, Wf.T, Wo.T, Wc.T], axis=1)
  ucat = jnp.concatenate([Ui.T, Uf.T, Uo.T, Uc.T], axis=1)
  bcat = jnp.concatenate([Ui_b, Uf_b, Uo_b, Uc_b])[None, :]

  blk = 512
  nblk = B // blk
  hncn, score2d = pl.pallas_call(
      _lstm_body,
      grid=(nblk,),
      in_specs=[
          pl.BlockSpec((blk, H), lambda i: (i, 0)),
          pl.BlockSpec((blk, 4 * H), lambda i: (i, 0)),
          pl.BlockSpec((blk, H), lambda i: (i, 0)),
          pl.BlockSpec((1, 1, blk), lambda i: (i, 0, 0)),
          pl.BlockSpec((H, 4 * H), lambda i: (0, 0)),
          pl.BlockSpec((H, 4 * H), lambda i: (0, 0)),
          pl.BlockSpec((1, 4 * H), lambda i: (0, 0)),
      ],
      out_specs=[
          pl.BlockSpec((blk, 4 * H), lambda i: (i, 0)),
          pl.BlockSpec((1, 1, blk), lambda i: (i, 0, 0)),
      ],
      out_shape=[
          jax.ShapeDtypeStruct((B, 4 * H), f32),
          jax.ShapeDtypeStruct((nblk, 1, blk), f32),
      ],
      name="tc_lstm_gates",
  )(x, hpcp, wr, wrb.reshape(nblk, 1, blk), wcat, ucat, bcat)
  score = score2d.reshape(B)

  lcap = wmax + L
  h_refT = jax.new_ref(hT)
  C_refT = jax.new_ref(CT)
  scatter = pl.kernel(
      functools.partial(_scatter_body, B, U),
      out_type=(),
      mesh=mesh,
      scratch_types=[
          pltpu.VMEM((seg,), i32),
          pltpu.VMEM((tr,), i32),
          pltpu.VMEM((lcap,), i32),
          pltpu.VMEM((lcap,), i32),
          pltpu.VMEM((CH,), i32),
          pltpu.VMEM((CH,), i32),
          pltpu.VMEM((H, wmax), f32),
          pltpu.VMEM((CH, 4 * H), f32),
          pltpu.VMEM((L,), i32),
          pltpu.SemaphoreType.DMA,
      ],
      compiler_params=pltpu.CompilerParams(needs_layout_passes=False,
                                           use_tc_tiling_on_sc=True,
                                           disable_bounds_checks=True),
      name="sc_lstm_scatter",
  )
  scatter(uidx, hncn, dayv + 1, h_refT, C_refT)
  h_out = jnp.transpose(jax.freeze(h_refT), (2, 1, 0))
  C_out = jnp.transpose(jax.freeze(C_refT), (2, 1, 0))
  return h_out, C_out, score


# speedup vs baseline: 10.2217x; 1.3535x over previous
import functools

import jax
import jax.numpy as jnp
from jax import lax
from jax.experimental import pallas as pl
from jax.experimental.pallas import tpu as pltpu
from jax.experimental.pallas import tpu_sc as plsc

NC = 2
NS = 16
NW = NC * NS
L = 16
CH = 128
H_LANES = 32


def _worker_id():
  return lax.axis_index("s") * NC + lax.axis_index("c")


def _iota():
  return lax.iota(jnp.int32, L)


def _gather_body(B, H, uidx_hbm, eidx_hbm, pidx_hbm, pemb_hbm, uemb_hbm,
                 wr_hbm, wrb_hbm, x_out, wr_out, wrb_out,
                 uidx_v, eidx_v, pidx_v, xp_v, xu_v, wrv_v, wrb_v, sem):
  ept = B // NW
  nch = ept // CH
  wid = _worker_id()
  base = wid * ept
  pltpu.sync_copy(uidx_hbm.at[pl.ds(base, ept)], uidx_v)
  pltpu.sync_copy(eidx_hbm.at[pl.ds(base, ept)], eidx_v)
  pltpu.sync_copy(pidx_hbm.at[pl.ds(base, ept)], pidx_v)

  def chunk(c, _):
    off = c * CH
    obase = base + off
    pltpu.async_copy(pemb_hbm.at[eidx_v.at[pl.ds(off, CH)]], xp_v, sem).wait()
    pltpu.async_copy(uemb_hbm.at[uidx_v.at[pl.ds(off, CH)]], xu_v, sem).wait()
    pltpu.async_copy(wr_hbm.at[pidx_v.at[pl.ds(off, CH)]], wrv_v, sem).wait()
    pltpu.async_copy(wrb_hbm.at[pidx_v.at[pl.ds(off, CH)]], wrb_v, sem).wait()

    def add_row(r, _):
      xp_v[r, pl.ds(0, L)] = xp_v[r, pl.ds(0, L)] + xu_v[r, pl.ds(0, L)]
      xp_v[r, pl.ds(L, L)] = xp_v[r, pl.ds(L, L)] + xu_v[r, pl.ds(L, L)]
      return 0

    lax.fori_loop(0, CH, add_row, 0)
    pltpu.sync_copy(xp_v, x_out.at[pl.ds(obase, CH)])
    pltpu.sync_copy(wrv_v, wr_out.at[pl.ds(obase, CH)])
    pltpu.sync_copy(wrb_v, wrb_out.at[pl.ds(obase, CH)])
    return 0

  lax.fori_loop(0, nch, chunk, 0)


def _state_gather_body(B, U, uidx_hbm, dayv_hbm, h_hbm, C_hbm, hpcp_out,
                       seg_v, evlist_v, uvlist_v, u128_v, ev128_v,
                       winh_v, winc_v, st_v, dayv_v, sem):
  tr, last = _tile_range(U)
  wid = _worker_id()
  lo = wid * tr
  pltpu.sync_copy(dayv_hbm, dayv_v)
  pltpu.sync_copy(uidx_hbm, seg_v)
  ds_ = jnp.max(dayv_v[...])
  iot = _iota()

  def do_windows(width, nsw):
    for sw in range(nsw):
      slo = lo + sw * width
      shi = jnp.minimum(slo + width, U)
      wdst_h = winh_v.at[:, pl.ds(0, width)]
      wdst_c = winc_v.at[:, pl.ds(0, width)]
      cp_h = pltpu.async_copy(h_hbm.at[ds_, :, pl.ds(slo, width)], wdst_h,
                              sem)
      cp_c = pltpu.async_copy(C_hbm.at[ds_, :, pl.ds(slo, width)], wdst_c,
                              sem)

      def scan(i, m):
        uv = seg_v[pl.ds(i * L, L)]
        mine = (uv >= slo) & (uv < shi)
        plsc.store_compressed(evlist_v.at[pl.ds(m, L)], i * L + iot,
                              mask=mine)
        plsc.store_compressed(uvlist_v.at[pl.ds(m, L)], uv - slo,
                              mask=mine)
        return m + jnp.max(plsc.all_reduce_population_count(mine))

      m_tot = lax.fori_loop(0, B // L, scan, jnp.int32(0))
      cp_h.wait()
      cp_c.wait()

      @pl.when(m_tot > 0)
      def _():
        nch = (m_tot + CH - 1) // CH

        def chunk(c, _):
          for s in range(CH // L):
            pos = jnp.minimum(c * CH + s * L + iot, m_tot - 1)
            u128_v[pl.ds(s * L, L)] = plsc.load_gather(uvlist_v, [pos])
            ev128_v[pl.ds(s * L, L)] = plsc.load_gather(evlist_v, [pos])

          def ext(e, _):
            ub = plsc.load_gather(u128_v, [jnp.full((L,), e, jnp.int32)])
            st_v[e, pl.ds(0, L)] = plsc.load_gather(winh_v, [iot, ub])
            st_v[e, pl.ds(L, L)] = plsc.load_gather(winh_v, [iot + L, ub])
            st_v[e, pl.ds(2 * L, L)] = plsc.load_gather(winc_v, [iot, ub])
            st_v[e, pl.ds(3 * L, L)] = plsc.load_gather(winc_v,
                                                        [iot + L, ub])
            return 0

          lax.fori_loop(0, CH, ext, 0)
          pltpu.async_copy(st_v, hpcp_out.at[ev128_v], sem).wait()
          return 0

        lax.fori_loop(0, nch, chunk, 0)

  last_pad = (last + 127) // 128 * 128

  @pl.when(wid < NW - 1)
  def _():
    do_windows(tr // 5, 5)

  @pl.when(wid == NW - 1)
  def _():
    do_windows(last_pad, 1)


def _tile_range(U):
  tr = (((U + NW - 1) // NW) + 127) // 128 * 128
  last = U - (NW - 1) * tr
  assert 0 < last <= tr
  return tr, last


def _scatter_body(B, U, uidx_hbm, hncn_hbm, day1v_hbm, h_ref, C_ref,
                  seg_v, w_v, wulist_v, welist_v, u128_v, ev128_v,
                  win_v, hv_v, day1_v, sem):
  tr, last = _tile_range(U)
  wid = _worker_id()
  lo = wid * tr
  hi = jnp.minimum(lo + tr, U)
  pltpu.sync_copy(day1v_hbm, day1_v)
  d1s = jnp.max(day1_v[...])
  iot = _iota()
  neg1 = jnp.full((L,), -1, jnp.int32)

  def winit(j, _):
    w_v[pl.ds(j * L, L)] = neg1
    return 0

  lax.fori_loop(0, tr // L, winit, 0)

  pltpu.sync_copy(uidx_hbm, seg_v)

  def wpass(i, _):
    uv = seg_v[pl.ds(i * L, L)]
    mine = (uv >= lo) & (uv < hi)
    rel = jnp.where(mine, uv - lo, 0)
    plsc.store_scatter(w_v, [rel], i * L + iot, mask=mine)
    return 0

  lax.fori_loop(0, B // L, wpass, 0)

  def do_windows(width, nsw):
    for sw in range(nsw):
      swbase = sw * width
      slo = lo + swbase

      def enum(j, m):
        wv = w_v[pl.ds(swbase + j * L, L)]
        has = wv >= 0
        plsc.store_compressed(wulist_v.at[pl.ds(m, L)], j * L + iot,
                              mask=has)
        plsc.store_compressed(welist_v.at[pl.ds(m, L)], wv, mask=has)
        return m + jnp.max(plsc.all_reduce_population_count(has))

      m_tot = lax.fori_loop(0, width // L, enum, jnp.int32(0))

      for ref, lane0 in ((h_ref, 0), (C_ref, H_LANES)):
        dst = win_v.at[:, pl.ds(0, width)]
        pltpu.async_copy(ref.at[d1s, :, pl.ds(slo, width)], dst, sem).wait()

        @pl.when(m_tot > 0)
        def _():
          nch = (m_tot + CH - 1) // CH

          def chunk(c, _):
            for s in range(CH // L):
              pos = jnp.minimum(c * CH + s * L + iot, m_tot - 1)
              u128_v[pl.ds(s * L, L)] = plsc.load_gather(wulist_v, [pos])
              ev128_v[pl.ds(s * L, L)] = plsc.load_gather(welist_v, [pos])
            pltpu.async_copy(hncn_hbm.at[ev128_v], hv_v, sem).wait()

            def ins(e, _):
              ub = plsc.load_gather(u128_v, [jnp.full((L,), e, jnp.int32)])
              plsc.store_scatter(win_v, [iot, ub], hv_v[e, pl.ds(lane0, L)])
              plsc.store_scatter(win_v, [iot + L, ub],
                                 hv_v[e, pl.ds(lane0 + L, L)])
              return 0

            lax.fori_loop(0, CH, ins, 0)
            return 0

          lax.fori_loop(0, nch, chunk, 0)

        pltpu.async_copy(dst, ref.at[d1s, :, pl.ds(slo, width)], sem).wait()

  last_pad = (last + 127) // 128 * 128

  @pl.when(wid < NW - 1)
  def _():
    do_windows(tr // 5, 5)

  @pl.when(wid == NW - 1)
  def _():
    do_windows(last_pad, 1)


def _copy_body(hi_ref, ci_ref, ho_ref, co_ref):
  ho_ref[...] = hi_ref[...]
  co_ref[...] = ci_ref[...]


def _lstm_body(x_ref, hpcp_ref, wr_ref, wrb_ref, wcat_ref, ucat_ref,
               bcat_ref, hncn_ref, sc_ref):
  x = x_ref[...]
  h = x.shape[1]
  hp = hpcp_ref[:, :h]
  cp = hpcp_ref[:, h:2 * h]
  g = (jnp.dot(x, wcat_ref[...], preferred_element_type=jnp.float32)
       + jnp.dot(hp, ucat_ref[...], preferred_element_type=jnp.float32)
       + bcat_ref[...])
  gi = jax.nn.sigmoid(g[:, :h])
  gf = jax.nn.sigmoid(g[:, h:2 * h])
  go = jax.nn.sigmoid(g[:, 2 * h:3 * h])
  gc = jnp.tanh(g[:, 3 * h:])
  cn = gf * cp + gi * gc
  hn = go * jnp.tanh(cn)
  pad = jnp.zeros((x.shape[0], 2 * h), jnp.float32)
  hncn_ref[...] = jnp.concatenate([hn, cn, pad], axis=1)
  s = jnp.sum(hn * wr_ref[...], axis=1)[None, None, :] + wrb_ref[...]
  sc_ref[...] = jax.nn.sigmoid(s)


def kernel(user_idx, emb_idx, problem_idx, day, h, C, user_emb, problem_emb,
           Wi, Wf, Wo, Wc, Ui, Uf, Uo, Uc, Ui_b, Uf_b, Uo_b, Uc_b, Wr, Wr_b):
  B = user_idx.shape[0]
  U, H, D1 = h.shape
  uidx = user_idx.astype(jnp.int32)
  eidx = emb_idx.astype(jnp.int32)
  pidx = problem_idx.astype(jnp.int32)
  dayv = jnp.full((L,), day, jnp.int32)
  mesh = plsc.VectorSubcoreMesh(core_axis_name="c", subcore_axis_name="s",
                                num_cores=NC, num_subcores=NS)
  sc_params = pltpu.CompilerParams(needs_layout_passes=False,
                                   use_tc_tiling_on_sc=False)

  f32 = jnp.float32
  i32 = jnp.int32
  ept = B // NW

  gather = pl.kernel(
      functools.partial(_gather_body, B, H),
      out_type=(
          jax.ShapeDtypeStruct((B, H), f32),
          jax.ShapeDtypeStruct((B, H), f32),
          jax.ShapeDtypeStruct((B,), f32),
      ),
      mesh=mesh,
      scratch_types=[
          pltpu.VMEM((ept,), i32),
          pltpu.VMEM((ept,), i32),
          pltpu.VMEM((ept,), i32),
          pltpu.VMEM((CH, H), f32),
          pltpu.VMEM((CH, H), f32),
          pltpu.VMEM((CH, H), f32),
          pltpu.VMEM((CH,), f32),
          pltpu.SemaphoreType.DMA,
      ],
      compiler_params=sc_params,
      name="sc_lstm_gather",
  )
  x, wr, wrb = gather(uidx, eidx, pidx, problem_emb, user_emb, Wr, Wr_b)

  tr, _last = _tile_range(U)
  seg = 4096
  wmax = max(tr // 5, (_last + 127) // 128 * 128)
  hT = jnp.transpose(h, (2, 1, 0))
  CT = jnp.transpose(C, (2, 1, 0))
  sc_tiled_params = pltpu.CompilerParams(needs_layout_passes=False,
                                         use_tc_tiling_on_sc=True,
                                         disable_bounds_checks=True)
  state_gather = pl.kernel(
      functools.partial(_state_gather_body, B, U),
      out_type=jax.ShapeDtypeStruct((B, 4 * H), f32),
      mesh=mesh,
      scratch_types=[
          pltpu.VMEM((B,), i32),
          pltpu.VMEM((B + L,), i32),
          pltpu.VMEM((B + L,), i32),
          pltpu.VMEM((CH,), i32),
          pltpu.VMEM((CH,), i32),
          pltpu.VMEM((H, wmax), f32),
          pltpu.VMEM((H, wmax), f32),
          pltpu.VMEM((CH, 4 * H), f32),
          pltpu.VMEM((L,), i32),
          pltpu.SemaphoreType.DMA,
      ],
      compiler_params=sc_tiled_params,
      name="sc_lstm_state_gather",
  )
  hpcp = state_gather(uidx, dayv, hT, CT)

  wcat = jnp.concatenate([Wi.T, Wf.T, Wo.T, Wc.T], axis=1)
  ucat = jnp.concatenate([Ui.T, Uf.T, Uo.T, Uc.T], axis=1)
  bcat = jnp.concatenate([Ui_b, Uf_b, Uo_b, Uc_b])[None, :]

  blk = 512
  nblk = B // blk
  hncn, score2d = pl.pallas_call(
      _lstm_body,
      grid=(nblk,),
      in_specs=[
          pl.BlockSpec((blk, H), lambda i: (i, 0)),
          pl.BlockSpec((blk, 4 * H), lambda i: (i, 0)),
          pl.BlockSpec((blk, H), lambda i: (i, 0)),
          pl.BlockSpec((1, 1, blk), lambda i: (i, 0, 0)),
          pl.BlockSpec((H, 4 * H), lambda i: (0, 0)),
          pl.BlockSpec((H, 4 * H), lambda i: (0, 0)),
          pl.BlockSpec((1, 4 * H), lambda i: (0, 0)),
      ],
      out_specs=[
          pl.BlockSpec((blk, 4 * H), lambda i: (i, 0)),
          pl.BlockSpec((1, 1, blk), lambda i: (i, 0, 0)),
      ],
      out_shape=[
          jax.ShapeDtypeStruct((B, 4 * H), f32),
          jax.ShapeDtypeStruct((nblk, 1, blk), f32),
      ],
      name="tc_lstm_gates",
  )(x, hpcp, wr, wrb.reshape(nblk, 1, blk), wcat, ucat, bcat)
  score = score2d.reshape(B)

  lcap = wmax + L
  cb = 8192
  ncb = (U + cb - 1) // cb
  hTc, CTc = pl.pallas_call(
      _copy_body,
      grid=(D1, ncb),
      in_specs=[
          pl.BlockSpec((1, H, cb), lambda i, j: (i, 0, j)),
          pl.BlockSpec((1, H, cb), lambda i, j: (i, 0, j)),
      ],
      out_specs=[
          pl.BlockSpec((1, H, cb), lambda i, j: (i, 0, j)),
          pl.BlockSpec((1, H, cb), lambda i, j: (i, 0, j)),
      ],
      out_shape=[
          jax.ShapeDtypeStruct((D1, H, U), f32),
          jax.ShapeDtypeStruct((D1, H, U), f32),
      ],
      name="tc_state_copy",
  )(hT, CT)
  h_refT = jax.new_ref(hTc)
  C_refT = jax.new_ref(CTc)
  scatter = pl.kernel(
      functools.partial(_scatter_body, B, U),
      out_type=(),
      mesh=mesh,
      scratch_types=[
          pltpu.VMEM((B,), i32),
          pltpu.VMEM((tr,), i32),
          pltpu.VMEM((lcap,), i32),
          pltpu.VMEM((lcap,), i32),
          pltpu.VMEM((CH,), i32),
          pltpu.VMEM((CH,), i32),
          pltpu.VMEM((H, wmax), f32),
          pltpu.VMEM((CH, 4 * H), f32),
          pltpu.VMEM((L,), i32),
          pltpu.SemaphoreType.DMA,
      ],
      compiler_params=pltpu.CompilerParams(needs_layout_passes=False,
                                           use_tc_tiling_on_sc=True,
                                           disable_bounds_checks=True),
      name="sc_lstm_scatter",
  )
  scatter(uidx, hncn, dayv + 1, h_refT, C_refT)
  h_out = jnp.transpose(jax.freeze(h_refT), (2, 1, 0))
  C_out = jnp.transpose(jax.freeze(C_refT), (2, 1, 0))
  return h_out, C_out, score


# speedup vs baseline: 10.7251x; 1.0493x over previous
import functools

import jax
import jax.numpy as jnp
from jax import lax
from jax.experimental import pallas as pl
from jax.experimental.pallas import tpu as pltpu
from jax.experimental.pallas import tpu_sc as plsc

NC = 2
NS = 16
NW = NC * NS
L = 16
CH = 128
H_LANES = 32


def _worker_id():
  return lax.axis_index("s") * NC + lax.axis_index("c")


def _iota():
  return lax.iota(jnp.int32, L)


def _gather_body(B, H, uidx_hbm, eidx_hbm, pidx_hbm, pemb_hbm, uemb_hbm,
                 wr_hbm, wrb_hbm, x_out, wr_out, wrb_out,
                 uidx_v, eidx_v, pidx_v, xp_v, xu_v, wrv_v, wrb_v, sem):
  ept = B // NW
  nch = ept // CH
  wid = _worker_id()
  base = wid * ept
  pltpu.sync_copy(uidx_hbm.at[pl.ds(base, ept)], uidx_v)
  pltpu.sync_copy(eidx_hbm.at[pl.ds(base, ept)], eidx_v)
  pltpu.sync_copy(pidx_hbm.at[pl.ds(base, ept)], pidx_v)

  def chunk(c, _):
    off = c * CH
    obase = base + off
    pltpu.async_copy(pemb_hbm.at[eidx_v.at[pl.ds(off, CH)]], xp_v, sem).wait()
    pltpu.async_copy(uemb_hbm.at[uidx_v.at[pl.ds(off, CH)]], xu_v, sem).wait()
    pltpu.async_copy(wr_hbm.at[pidx_v.at[pl.ds(off, CH)]], wrv_v, sem).wait()
    pltpu.async_copy(wrb_hbm.at[pidx_v.at[pl.ds(off, CH)]], wrb_v, sem).wait()

    def add_row(r, _):
      xp_v[r, pl.ds(0, L)] = xp_v[r, pl.ds(0, L)] + xu_v[r, pl.ds(0, L)]
      xp_v[r, pl.ds(L, L)] = xp_v[r, pl.ds(L, L)] + xu_v[r, pl.ds(L, L)]
      return 0

    lax.fori_loop(0, CH, add_row, 0)
    pltpu.sync_copy(xp_v, x_out.at[pl.ds(obase, CH)])
    pltpu.sync_copy(wrv_v, wr_out.at[pl.ds(obase, CH)])
    pltpu.sync_copy(wrb_v, wrb_out.at[pl.ds(obase, CH)])
    return 0

  lax.fori_loop(0, nch, chunk, 0)


def _state_gather_body(B, U, uidx_hbm, dayv_hbm, h_hbm, C_hbm, hpcp_out,
                       seg_v, evlist_v, uvlist_v, u128_v, ev128_v,
                       winh_v, winc_v, st_v, dayv_v, sem):
  tr, last = _tile_range(U)
  wid = _worker_id()
  lo = wid * tr
  pltpu.sync_copy(dayv_hbm, dayv_v)
  pltpu.sync_copy(uidx_hbm, seg_v)
  ds_ = jnp.max(dayv_v[...])
  iot = _iota()

  def do_windows(width, nsw):
    for sw in range(nsw):
      slo = lo + sw * width
      shi = jnp.minimum(slo + width, U)
      wdst_h = winh_v.at[:, pl.ds(0, width)]
      wdst_c = winc_v.at[:, pl.ds(0, width)]
      cp_h = pltpu.async_copy(h_hbm.at[ds_, :, pl.ds(slo, width)], wdst_h,
                              sem)
      cp_c = pltpu.async_copy(C_hbm.at[ds_, :, pl.ds(slo, width)], wdst_c,
                              sem)

      def scan(i, m):
        uv = seg_v[pl.ds(i * L, L)]
        mine = (uv >= slo) & (uv < shi)
        plsc.store_compressed(evlist_v.at[pl.ds(m, L)], i * L + iot,
                              mask=mine)
        plsc.store_compressed(uvlist_v.at[pl.ds(m, L)], uv - slo,
                              mask=mine)
        return m + jnp.max(plsc.all_reduce_population_count(mine))

      m_tot = lax.fori_loop(0, B // L, scan, jnp.int32(0))
      cp_h.wait()
      cp_c.wait()

      @pl.when(m_tot > 0)
      def _():
        nch = (m_tot + CH - 1) // CH

        def chunk(c, _):
          for s in range(CH // L):
            pos = jnp.minimum(c * CH + s * L + iot, m_tot - 1)
            u128_v[pl.ds(s * L, L)] = plsc.load_gather(uvlist_v, [pos])
            ev128_v[pl.ds(s * L, L)] = plsc.load_gather(evlist_v, [pos])

          def ext(e, _):
            ub = plsc.load_gather(u128_v, [jnp.full((L,), e, jnp.int32)])
            st_v[e, pl.ds(0, L)] = plsc.load_gather(winh_v, [iot, ub])
            st_v[e, pl.ds(L, L)] = plsc.load_gather(winh_v, [iot + L, ub])
            st_v[e, pl.ds(2 * L, L)] = plsc.load_gather(winc_v, [iot, ub])
            st_v[e, pl.ds(3 * L, L)] = plsc.load_gather(winc_v,
                                                        [iot + L, ub])
            return 0

          lax.fori_loop(0, CH, ext, 0)
          pltpu.async_copy(st_v, hpcp_out.at[ev128_v], sem).wait()
          return 0

        lax.fori_loop(0, nch, chunk, 0)

  last_pad = (last + 127) // 128 * 128

  @pl.when(wid < NW - 1)
  def _():
    do_windows(tr // 5, 5)

  @pl.when(wid == NW - 1)
  def _():
    do_windows(last_pad, 1)


def _tile_range(U):
  tr = (((U + NW - 1) // NW) + 127) // 128 * 128
  last = U - (NW - 1) * tr
  assert 0 < last <= tr
  return tr, last


def _scatter_body(B, U, uidx_hbm, hncn_hbm, day1v_hbm, h_ref, C_ref,
                  seg_v, w_v, wulist_v, welist_v, u128_v, ev128_v,
                  win_v, hv_v, day1_v, sem):
  tr, last = _tile_range(U)
  wid = _worker_id()
  lo = wid * tr
  hi = jnp.minimum(lo + tr, U)
  pltpu.sync_copy(day1v_hbm, day1_v)
  d1s = jnp.max(day1_v[...])
  iot = _iota()
  neg1 = jnp.full((L,), -1, jnp.int32)

  def winit(j, _):
    w_v[pl.ds(j * L, L)] = neg1
    return 0

  lax.fori_loop(0, tr // L, winit, 0)

  pltpu.sync_copy(uidx_hbm, seg_v)

  def wpass(i, _):
    uv = seg_v[pl.ds(i * L, L)]
    mine = (uv >= lo) & (uv < hi)
    rel = jnp.where(mine, uv - lo, 0)
    plsc.store_scatter(w_v, [rel], i * L + iot, mask=mine)
    return 0

  lax.fori_loop(0, B // L, wpass, 0)

  def do_windows(width, nsw):
    for sw in range(nsw):
      swbase = sw * width
      slo = lo + swbase

      def enum(j, m):
        wv = w_v[pl.ds(swbase + j * L, L)]
        has = wv >= 0
        plsc.store_compressed(wulist_v.at[pl.ds(m, L)], j * L + iot,
                              mask=has)
        plsc.store_compressed(welist_v.at[pl.ds(m, L)], wv, mask=has)
        return m + jnp.max(plsc.all_reduce_population_count(has))

      m_tot = lax.fori_loop(0, width // L, enum, jnp.int32(0))

      for ref, lane0 in ((h_ref, 0), (C_ref, H_LANES)):
        dst = win_v.at[:, pl.ds(0, width)]
        pltpu.async_copy(ref.at[d1s, :, pl.ds(slo, width)], dst, sem).wait()

        @pl.when(m_tot > 0)
        def _():
          nch = (m_tot + CH - 1) // CH

          def chunk(c, _):
            for s in range(CH // L):
              pos = jnp.minimum(c * CH + s * L + iot, m_tot - 1)
              u128_v[pl.ds(s * L, L)] = plsc.load_gather(wulist_v, [pos])
              ev128_v[pl.ds(s * L, L)] = plsc.load_gather(welist_v, [pos])
            pltpu.async_copy(hncn_hbm.at[ev128_v], hv_v, sem).wait()

            def ins(e, _):
              ub = plsc.load_gather(u128_v, [jnp.full((L,), e, jnp.int32)])
              plsc.store_scatter(win_v, [iot, ub], hv_v[e, pl.ds(lane0, L)])
              plsc.store_scatter(win_v, [iot + L, ub],
                                 hv_v[e, pl.ds(lane0 + L, L)])
              return 0

            lax.fori_loop(0, CH, ins, 0)
            return 0

          lax.fori_loop(0, nch, chunk, 0)

        pltpu.async_copy(dst, ref.at[d1s, :, pl.ds(slo, width)], sem).wait()

  last_pad = (last + 127) // 128 * 128

  @pl.when(wid < NW - 1)
  def _():
    do_windows(tr // 5, 5)

  @pl.when(wid == NW - 1)
  def _():
    do_windows(last_pad, 1)


def _copy_body(hi_ref, ci_ref, ho_ref, co_ref):
  ho_ref[...] = hi_ref[...]
  co_ref[...] = ci_ref[...]


def _lstm_body(x_ref, hpcp_ref, wr_ref, wrb_ref, wcat_ref, ucat_ref,
               bcat_ref, hncn_ref, sc_ref):
  x = x_ref[...]
  h = x.shape[1]
  hp = hpcp_ref[:, :h]
  cp = hpcp_ref[:, h:2 * h]
  g = (jnp.dot(x, wcat_ref[...], preferred_element_type=jnp.float32)
       + jnp.dot(hp, ucat_ref[...], preferred_element_type=jnp.float32)
       + bcat_ref[...])
  gi = jax.nn.sigmoid(g[:, :h])
  gf = jax.nn.sigmoid(g[:, h:2 * h])
  go = jax.nn.sigmoid(g[:, 2 * h:3 * h])
  gc = jnp.tanh(g[:, 3 * h:])
  cn = gf * cp + gi * gc
  hn = go * jnp.tanh(cn)
  pad = jnp.zeros((x.shape[0], 2 * h), jnp.float32)
  hncn_ref[...] = jnp.concatenate([hn, cn, pad], axis=1)
  s = jnp.sum(hn * wr_ref[...], axis=1)[None, None, :] + wrb_ref[...]
  sc_ref[...] = jax.nn.sigmoid(s)


def kernel(user_idx, emb_idx, problem_idx, day, h, C, user_emb, problem_emb,
           Wi, Wf, Wo, Wc, Ui, Uf, Uo, Uc, Ui_b, Uf_b, Uo_b, Uc_b, Wr, Wr_b):
  B = user_idx.shape[0]
  U, H, D1 = h.shape
  uidx = user_idx.astype(jnp.int32)
  eidx = emb_idx.astype(jnp.int32)
  pidx = problem_idx.astype(jnp.int32)
  dayv = jnp.full((L,), day, jnp.int32)
  mesh = plsc.VectorSubcoreMesh(core_axis_name="c", subcore_axis_name="s",
                                num_cores=NC, num_subcores=NS)
  sc_params = pltpu.CompilerParams(needs_layout_passes=False,
                                   use_tc_tiling_on_sc=False)

  f32 = jnp.float32
  i32 = jnp.int32
  ept = B // NW

  gather = pl.kernel(
      functools.partial(_gather_body, B, H),
      out_type=(
          jax.ShapeDtypeStruct((B, H), f32),
          jax.ShapeDtypeStruct((B, H), f32),
          jax.ShapeDtypeStruct((B,), f32),
      ),
      mesh=mesh,
      scratch_types=[
          pltpu.VMEM((ept,), i32),
          pltpu.VMEM((ept,), i32),
          pltpu.VMEM((ept,), i32),
          pltpu.VMEM((CH, H), f32),
          pltpu.VMEM((CH, H), f32),
          pltpu.VMEM((CH, H), f32),
          pltpu.VMEM((CH,), f32),
          pltpu.SemaphoreType.DMA,
      ],
      compiler_params=sc_params,
      name="sc_lstm_gather",
  )
  x, wr, wrb = gather(uidx, eidx, pidx, problem_emb, user_emb, Wr, Wr_b)

  tr, _last = _tile_range(U)
  seg = 4096
  wmax = max(tr // 5, (_last + 127) // 128 * 128)
  hT = jnp.transpose(h, (2, 1, 0))
  CT = jnp.transpose(C, (2, 1, 0))
  sc_tiled_params = pltpu.CompilerParams(needs_layout_passes=False,
                                         use_tc_tiling_on_sc=True,
                                         disable_bounds_checks=True)
  state_gather = pl.kernel(
      functools.partial(_state_gather_body, B, U),
      out_type=jax.ShapeDtypeStruct((B, 4 * H), f32),
      mesh=mesh,
      scratch_types=[
          pltpu.VMEM((B,), i32),
          pltpu.VMEM((B + L,), i32),
          pltpu.VMEM((B + L,), i32),
          pltpu.VMEM((CH,), i32),
          pltpu.VMEM((CH,), i32),
          pltpu.VMEM((H, wmax), f32),
          pltpu.VMEM((H, wmax), f32),
          pltpu.VMEM((CH, 4 * H), f32),
          pltpu.VMEM((L,), i32),
          pltpu.SemaphoreType.DMA,
      ],
      compiler_params=sc_tiled_params,
      name="sc_lstm_state_gather",
  )
  hpcp = state_gather(uidx, dayv, hT, CT)

  wcat = jnp.concatenate([Wi.T, Wf.T, Wo.T, Wc.T], axis=1)
  ucat = jnp.concatenate([Ui.T, Uf.T, Uo.T, Uc.T], axis=1)
  bcat = jnp.concatenate([Ui_b, Uf_b, Uo_b, Uc_b])[None, :]

  blk = 512
  nblk = B // blk
  hncn, score2d = pl.pallas_call(
      _lstm_body,
      grid=(nblk,),
      in_specs=[
          pl.BlockSpec((blk, H), lambda i: (i, 0)),
          pl.BlockSpec((blk, 4 * H), lambda i: (i, 0)),
          pl.BlockSpec((blk, H), lambda i: (i, 0)),
          pl.BlockSpec((1, 1, blk), lambda i: (i, 0, 0)),
          pl.BlockSpec((H, 4 * H), lambda i: (0, 0)),
          pl.BlockSpec((H, 4 * H), lambda i: (0, 0)),
          pl.BlockSpec((1, 4 * H), lambda i: (0, 0)),
      ],
      out_specs=[
          pl.BlockSpec((blk, 4 * H), lambda i: (i, 0)),
          pl.BlockSpec((1, 1, blk), lambda i: (i, 0, 0)),
      ],
      out_shape=[
          jax.ShapeDtypeStruct((B, 4 * H), f32),
          jax.ShapeDtypeStruct((nblk, 1, blk), f32),
      ],
      name="tc_lstm_gates",
  )(x, hpcp, wr, wrb.reshape(nblk, 1, blk), wcat, ucat, bcat)
  score = score2d.reshape(B)

  lcap = wmax + L
  cb = 20096
  ncb = (U + cb - 1) // cb
  hTc, CTc = pl.pallas_call(
      _copy_body,
      grid=(D1, ncb),
      in_specs=[
          pl.BlockSpec((1, H, cb), lambda i, j: (i, 0, j)),
          pl.BlockSpec((1, H, cb), lambda i, j: (i, 0, j)),
      ],
      out_specs=[
          pl.BlockSpec((1, H, cb), lambda i, j: (i, 0, j)),
          pl.BlockSpec((1, H, cb), lambda i, j: (i, 0, j)),
      ],
      out_shape=[
          jax.ShapeDtypeStruct((D1, H, U), f32),
          jax.ShapeDtypeStruct((D1, H, U), f32),
      ],
      name="tc_state_copy",
  )(hT, CT)
  h_refT = jax.new_ref(hTc)
  C_refT = jax.new_ref(CTc)
  scatter = pl.kernel(
      functools.partial(_scatter_body, B, U),
      out_type=(),
      mesh=mesh,
      scratch_types=[
          pltpu.VMEM((B,), i32),
          pltpu.VMEM((tr,), i32),
          pltpu.VMEM((lcap,), i32),
          pltpu.VMEM((lcap,), i32),
          pltpu.VMEM((CH,), i32),
          pltpu.VMEM((CH,), i32),
          pltpu.VMEM((H, wmax), f32),
          pltpu.VMEM((CH, 4 * H), f32),
          pltpu.VMEM((L,), i32),
          pltpu.SemaphoreType.DMA,
      ],
      compiler_params=pltpu.CompilerParams(needs_layout_passes=False,
                                           use_tc_tiling_on_sc=True,
                                           disable_bounds_checks=True),
      name="sc_lstm_scatter",
  )
  scatter(uidx, hncn, dayv + 1, h_refT, C_refT)
  h_out = jnp.transpose(jax.freeze(h_refT), (2, 1, 0))
  C_out = jnp.transpose(jax.freeze(C_refT), (2, 1, 0))
  return h_out, C_out, score


# speedup vs baseline: 10.8097x; 1.0079x over previous
import functools

import jax
import jax.numpy as jnp
from jax import lax
from jax.experimental import pallas as pl
from jax.experimental.pallas import tpu as pltpu
from jax.experimental.pallas import tpu_sc as plsc

NC = 2
NS = 16
NW = NC * NS
L = 16
CH = 128
H_LANES = 32


def _worker_id():
  return lax.axis_index("s") * NC + lax.axis_index("c")


def _iota():
  return lax.iota(jnp.int32, L)


def _gather_body(B, H, uidx_hbm, eidx_hbm, pidx_hbm, pemb_hbm, uemb_hbm,
                 wr_hbm, wrb_hbm, x_out, wr_out, wrb_out,
                 uidx_v, eidx_v, pidx_v, xp_v, xu_v, wrv_v, wrb_v, sem):
  ept = B // NW
  nch = ept // CH
  wid = _worker_id()
  base = wid * ept
  pltpu.sync_copy(uidx_hbm.at[pl.ds(base, ept)], uidx_v)
  pltpu.sync_copy(eidx_hbm.at[pl.ds(base, ept)], eidx_v)
  pltpu.sync_copy(pidx_hbm.at[pl.ds(base, ept)], pidx_v)

  def chunk(c, _):
    off = c * CH
    obase = base + off
    pltpu.async_copy(pemb_hbm.at[eidx_v.at[pl.ds(off, CH)]], xp_v, sem).wait()
    pltpu.async_copy(uemb_hbm.at[uidx_v.at[pl.ds(off, CH)]], xu_v, sem).wait()
    pltpu.async_copy(wr_hbm.at[pidx_v.at[pl.ds(off, CH)]], wrv_v, sem).wait()
    pltpu.async_copy(wrb_hbm.at[pidx_v.at[pl.ds(off, CH)]], wrb_v, sem).wait()

    def add_row(r, _):
      xp_v[r, pl.ds(0, L)] = xp_v[r, pl.ds(0, L)] + xu_v[r, pl.ds(0, L)]
      xp_v[r, pl.ds(L, L)] = xp_v[r, pl.ds(L, L)] + xu_v[r, pl.ds(L, L)]
      return 0

    lax.fori_loop(0, CH, add_row, 0)
    pltpu.sync_copy(xp_v, x_out.at[pl.ds(obase, CH)])
    pltpu.sync_copy(wrv_v, wr_out.at[pl.ds(obase, CH)])
    pltpu.sync_copy(wrb_v, wrb_out.at[pl.ds(obase, CH)])
    return 0

  lax.fori_loop(0, nch, chunk, 0)


def _state_gather_body(B, U, uidx_hbm, dayv_hbm, h_hbm, C_hbm, hpcp_out,
                       seg_v, evlist_v, uvlist_v, u128_v, ev128_v,
                       winh_v, winc_v, st_v, dayv_v, sem):
  tr, last = _tile_range(U)
  wid = _worker_id()
  lo = wid * tr
  pltpu.sync_copy(dayv_hbm, dayv_v)
  pltpu.sync_copy(uidx_hbm, seg_v)
  ds_ = jnp.max(dayv_v[...])
  iot = _iota()

  def do_windows(width, nsw):
    for sw in range(nsw):
      slo = lo + sw * width
      shi = jnp.minimum(slo + width, U)
      wdst_h = winh_v.at[:, pl.ds(0, width)]
      wdst_c = winc_v.at[:, pl.ds(0, width)]
      cp_h = pltpu.async_copy(h_hbm.at[ds_, :, pl.ds(slo, width)], wdst_h,
                              sem)
      cp_c = pltpu.async_copy(C_hbm.at[ds_, :, pl.ds(slo, width)], wdst_c,
                              sem)

      def scan(i, m):
        uv = seg_v[pl.ds(i * L, L)]
        mine = (uv >= slo) & (uv < shi)
        plsc.store_compressed(evlist_v.at[pl.ds(m, L)], i * L + iot,
                              mask=mine)
        plsc.store_compressed(uvlist_v.at[pl.ds(m, L)], uv - slo,
                              mask=mine)
        return m + jnp.max(plsc.all_reduce_population_count(mine))

      m_tot = lax.fori_loop(0, B // L, scan, jnp.int32(0))
      cp_h.wait()
      cp_c.wait()

      @pl.when(m_tot > 0)
      def _():
        nch = (m_tot + CH - 1) // CH

        def chunk(c, _):
          for s in range(CH // L):
            pos = jnp.minimum(c * CH + s * L + iot, m_tot - 1)
            u128_v[pl.ds(s * L, L)] = plsc.load_gather(uvlist_v, [pos])
            ev128_v[pl.ds(s * L, L)] = plsc.load_gather(evlist_v, [pos])

          def ext(e, _):
            ub = plsc.load_gather(u128_v, [jnp.full((L,), e, jnp.int32)])
            st_v[e, pl.ds(0, L)] = plsc.load_gather(winh_v, [iot, ub])
            st_v[e, pl.ds(L, L)] = plsc.load_gather(winh_v, [iot + L, ub])
            st_v[e, pl.ds(2 * L, L)] = plsc.load_gather(winc_v, [iot, ub])
            st_v[e, pl.ds(3 * L, L)] = plsc.load_gather(winc_v,
                                                        [iot + L, ub])
            return 0

          lax.fori_loop(0, CH, ext, 0)
          pltpu.async_copy(st_v, hpcp_out.at[ev128_v], sem).wait()
          return 0

        lax.fori_loop(0, nch, chunk, 0)

  last_pad = (last + 127) // 128 * 128

  @pl.when(wid < NW - 1)
  def _():
    do_windows(tr // 5, 5)

  @pl.when(wid == NW - 1)
  def _():
    do_windows(last_pad, 1)


def _tile_range(U):
  tr = (((U + NW - 1) // NW) + 127) // 128 * 128
  last = U - (NW - 1) * tr
  assert 0 < last <= tr
  return tr, last


def _scatter_body(B, U, uidx_hbm, hncn_hbm, day1v_hbm, h_ref, C_ref,
                  seg_v, w_v, wulist_v, welist_v, u128_v, ev128_v,
                  win_v, hv_v, day1_v, sem):
  tr, last = _tile_range(U)
  wid = _worker_id()
  lo = wid * tr
  hi = jnp.minimum(lo + tr, U)
  pltpu.sync_copy(day1v_hbm, day1_v)
  d1s = jnp.max(day1_v[...])
  iot = _iota()
  neg1 = jnp.full((L,), -1, jnp.int32)

  def winit(j, _):
    w_v[pl.ds(j * L, L)] = neg1
    return 0

  lax.fori_loop(0, tr // L, winit, 0)

  pltpu.sync_copy(uidx_hbm, seg_v)

  def wpass(i, _):
    uv = seg_v[pl.ds(i * L, L)]
    mine = (uv >= lo) & (uv < hi)
    rel = jnp.where(mine, uv - lo, 0)
    plsc.store_scatter(w_v, [rel], i * L + iot, mask=mine)
    return 0

  lax.fori_loop(0, B // L, wpass, 0)

  def do_windows(width, nsw):
    for sw in range(nsw):
      swbase = sw * width
      slo = lo + swbase

      def enum(j, m):
        wv = w_v[pl.ds(swbase + j * L, L)]
        has = wv >= 0
        plsc.store_compressed(wulist_v.at[pl.ds(m, L)], j * L + iot,
                              mask=has)
        plsc.store_compressed(welist_v.at[pl.ds(m, L)], wv, mask=has)
        return m + jnp.max(plsc.all_reduce_population_count(has))

      m_tot = lax.fori_loop(0, width // L, enum, jnp.int32(0))

      for ref, lane0 in ((h_ref, 0), (C_ref, H_LANES)):
        dst = win_v.at[:, pl.ds(0, width)]
        pltpu.async_copy(ref.at[d1s, :, pl.ds(slo, width)], dst, sem).wait()

        @pl.when(m_tot > 0)
        def _():
          nch = (m_tot + CH - 1) // CH

          def chunk(c, _):
            for s in range(CH // L):
              pos = jnp.minimum(c * CH + s * L + iot, m_tot - 1)
              u128_v[pl.ds(s * L, L)] = plsc.load_gather(wulist_v, [pos])
              ev128_v[pl.ds(s * L, L)] = plsc.load_gather(welist_v, [pos])
            pltpu.async_copy(hncn_hbm.at[ev128_v], hv_v, sem).wait()

            def ins(e, _):
              ub = plsc.load_gather(u128_v, [jnp.full((L,), e, jnp.int32)])
              plsc.store_scatter(win_v, [iot, ub], hv_v[e, pl.ds(lane0, L)])
              plsc.store_scatter(win_v, [iot + L, ub],
                                 hv_v[e, pl.ds(lane0 + L, L)])
              return 0

            lax.fori_loop(0, CH, ins, 0)
            return 0

          lax.fori_loop(0, nch, chunk, 0)

        pltpu.async_copy(dst, ref.at[d1s, :, pl.ds(slo, width)], sem).wait()

  last_pad = (last + 127) // 128 * 128

  @pl.when(wid < NW - 1)
  def _():
    do_windows(tr // 5, 5)

  @pl.when(wid == NW - 1)
  def _():
    do_windows(last_pad, 1)


def _copy_body(hi_ref, ci_ref, ho_ref, co_ref):
  ho_ref[...] = hi_ref[...]
  co_ref[...] = ci_ref[...]


def _lstm_body(x_ref, hpcp_ref, wr_ref, wrb_ref, wcat_ref, ucat_ref,
               bcat_ref, hncn_ref, sc_ref):
  x = x_ref[...]
  h = x.shape[1]
  hp = hpcp_ref[:, :h]
  cp = hpcp_ref[:, h:2 * h]
  g = (jnp.dot(x, wcat_ref[...], preferred_element_type=jnp.float32)
       + jnp.dot(hp, ucat_ref[...], preferred_element_type=jnp.float32)
       + bcat_ref[...])
  gi = jax.nn.sigmoid(g[:, :h])
  gf = jax.nn.sigmoid(g[:, h:2 * h])
  go = jax.nn.sigmoid(g[:, 2 * h:3 * h])
  gc = jnp.tanh(g[:, 3 * h:])
  cn = gf * cp + gi * gc
  hn = go * jnp.tanh(cn)
  pad = jnp.zeros((x.shape[0], 2 * h), jnp.float32)
  hncn_ref[...] = jnp.concatenate([hn, cn, pad], axis=1)
  s = jnp.sum(hn * wr_ref[...], axis=1)[None, None, :] + wrb_ref[...]
  sc_ref[...] = jax.nn.sigmoid(s)


def kernel(user_idx, emb_idx, problem_idx, day, h, C, user_emb, problem_emb,
           Wi, Wf, Wo, Wc, Ui, Uf, Uo, Uc, Ui_b, Uf_b, Uo_b, Uc_b, Wr, Wr_b):
  B = user_idx.shape[0]
  U, H, D1 = h.shape
  uidx = user_idx.astype(jnp.int32)
  eidx = emb_idx.astype(jnp.int32)
  pidx = problem_idx.astype(jnp.int32)
  dayv = jnp.full((L,), day, jnp.int32)
  mesh = plsc.VectorSubcoreMesh(core_axis_name="c", subcore_axis_name="s",
                                num_cores=NC, num_subcores=NS)
  sc_params = pltpu.CompilerParams(needs_layout_passes=False,
                                   use_tc_tiling_on_sc=False)

  f32 = jnp.float32
  i32 = jnp.int32
  ept = B // NW

  gather = pl.kernel(
      functools.partial(_gather_body, B, H),
      out_type=(
          jax.ShapeDtypeStruct((B, H), f32),
          jax.ShapeDtypeStruct((B, H), f32),
          jax.ShapeDtypeStruct((B,), f32),
      ),
      mesh=mesh,
      scratch_types=[
          pltpu.VMEM((ept,), i32),
          pltpu.VMEM((ept,), i32),
          pltpu.VMEM((ept,), i32),
          pltpu.VMEM((CH, H), f32),
          pltpu.VMEM((CH, H), f32),
          pltpu.VMEM((CH, H), f32),
          pltpu.VMEM((CH,), f32),
          pltpu.SemaphoreType.DMA,
      ],
      compiler_params=sc_params,
      name="sc_lstm_gather",
  )
  x, wr, wrb = gather(uidx, eidx, pidx, problem_emb, user_emb, Wr, Wr_b)

  tr, _last = _tile_range(U)
  seg = 4096
  wmax = max(tr // 5, (_last + 127) // 128 * 128)
  hT = jnp.transpose(h, (2, 1, 0))
  CT = jnp.transpose(C, (2, 1, 0))
  sc_tiled_params = pltpu.CompilerParams(needs_layout_passes=False,
                                         use_tc_tiling_on_sc=True,
                                         disable_bounds_checks=True)
  state_gather = pl.kernel(
      functools.partial(_state_gather_body, B, U),
      out_type=jax.ShapeDtypeStruct((B, 4 * H), f32),
      mesh=mesh,
      scratch_types=[
          pltpu.VMEM((B,), i32),
          pltpu.VMEM((B + L,), i32),
          pltpu.VMEM((B + L,), i32),
          pltpu.VMEM((CH,), i32),
          pltpu.VMEM((CH,), i32),
          pltpu.VMEM((H, wmax), f32),
          pltpu.VMEM((H, wmax), f32),
          pltpu.VMEM((CH, 4 * H), f32),
          pltpu.VMEM((L,), i32),
          pltpu.SemaphoreType.DMA,
      ],
      compiler_params=sc_tiled_params,
      name="sc_lstm_state_gather",
  )
  hpcp = state_gather(uidx, dayv, hT, CT)

  wcat = jnp.concatenate([Wi.T, Wf.T, Wo.T, Wc.T], axis=1)
  ucat = jnp.concatenate([Ui.T, Uf.T, Uo.T, Uc.T], axis=1)
  bcat = jnp.concatenate([Ui_b, Uf_b, Uo_b, Uc_b])[None, :]

  blk = 512
  nblk = B // blk
  hncn, score2d = pl.pallas_call(
      _lstm_body,
      grid=(nblk,),
      in_specs=[
          pl.BlockSpec((blk, H), lambda i: (i, 0)),
          pl.BlockSpec((blk, 4 * H), lambda i: (i, 0)),
          pl.BlockSpec((blk, H), lambda i: (i, 0)),
          pl.BlockSpec((1, 1, blk), lambda i: (i, 0, 0)),
          pl.BlockSpec((H, 4 * H), lambda i: (0, 0)),
          pl.BlockSpec((H, 4 * H), lambda i: (0, 0)),
          pl.BlockSpec((1, 4 * H), lambda i: (0, 0)),
      ],
      out_specs=[
          pl.BlockSpec((blk, 4 * H), lambda i: (i, 0)),
          pl.BlockSpec((1, 1, blk), lambda i: (i, 0, 0)),
      ],
      out_shape=[
          jax.ShapeDtypeStruct((B, 4 * H), f32),
          jax.ShapeDtypeStruct((nblk, 1, blk), f32),
      ],
      name="tc_lstm_gates",
  )(x, hpcp, wr, wrb.reshape(nblk, 1, blk), wcat, ucat, bcat)
  score = score2d.reshape(B)

  lcap = wmax + L
  cb = 50048
  ncb = (U + cb - 1) // cb
  hTc, CTc = pl.pallas_call(
      _copy_body,
      grid=(D1, ncb),
      in_specs=[
          pl.BlockSpec((1, H, cb), lambda i, j: (i, 0, j)),
          pl.BlockSpec((1, H, cb), lambda i, j: (i, 0, j)),
      ],
      out_specs=[
          pl.BlockSpec((1, H, cb), lambda i, j: (i, 0, j)),
          pl.BlockSpec((1, H, cb), lambda i, j: (i, 0, j)),
      ],
      out_shape=[
          jax.ShapeDtypeStruct((D1, H, U), f32),
          jax.ShapeDtypeStruct((D1, H, U), f32),
      ],
      name="tc_state_copy",
  )(hT, CT)
  h_refT = jax.new_ref(hTc)
  C_refT = jax.new_ref(CTc)
  scatter = pl.kernel(
      functools.partial(_scatter_body, B, U),
      out_type=(),
      mesh=mesh,
      scratch_types=[
          pltpu.VMEM((B,), i32),
          pltpu.VMEM((tr,), i32),
          pltpu.VMEM((lcap,), i32),
          pltpu.VMEM((lcap,), i32),
          pltpu.VMEM((CH,), i32),
          pltpu.VMEM((CH,), i32),
          pltpu.VMEM((H, wmax), f32),
          pltpu.VMEM((CH, 4 * H), f32),
          pltpu.VMEM((L,), i32),
          pltpu.SemaphoreType.DMA,
      ],
      compiler_params=pltpu.CompilerParams(needs_layout_passes=False,
                                           use_tc_tiling_on_sc=True,
                                           disable_bounds_checks=True),
      name="sc_lstm_scatter",
  )
  scatter(uidx, hncn, dayv + 1, h_refT, C_refT)
  h_out = jnp.transpose(jax.freeze(h_refT), (2, 1, 0))
  C_out = jnp.transpose(jax.freeze(C_refT), (2, 1, 0))
  return h_out, C_out, score


# speedup vs baseline: 11.3631x; 1.0512x over previous
import functools

import jax
import jax.numpy as jnp
from jax import lax
from jax.experimental import pallas as pl
from jax.experimental.pallas import tpu as pltpu
from jax.experimental.pallas import tpu_sc as plsc

NC = 2
NS = 16
NW = NC * NS
L = 16
CH = 128
H_LANES = 32


def _worker_id():
  return lax.axis_index("s") * NC + lax.axis_index("c")


def _iota():
  return lax.iota(jnp.int32, L)


def _gather_body(B, H, uidx_hbm, eidx_hbm, pidx_hbm, pemb_hbm, uemb_hbm,
                 wr_hbm, wrb_hbm, x_out, wr_out, wrb_out,
                 uidx_v, eidx_v, pidx_v, xp_v, xu_v, wrv_v, wrb_v, sem):
  ept = B // NW
  nch = ept // CH
  wid = _worker_id()
  base = wid * ept
  pltpu.sync_copy(uidx_hbm.at[pl.ds(base, ept)], uidx_v)
  pltpu.sync_copy(eidx_hbm.at[pl.ds(base, ept)], eidx_v)
  pltpu.sync_copy(pidx_hbm.at[pl.ds(base, ept)], pidx_v)

  def chunk(c, _):
    off = c * CH
    obase = base + off
    pltpu.async_copy(pemb_hbm.at[eidx_v.at[pl.ds(off, CH)]], xp_v, sem).wait()
    pltpu.async_copy(uemb_hbm.at[uidx_v.at[pl.ds(off, CH)]], xu_v, sem).wait()
    pltpu.async_copy(wr_hbm.at[pidx_v.at[pl.ds(off, CH)]], wrv_v, sem).wait()
    pltpu.async_copy(wrb_hbm.at[pidx_v.at[pl.ds(off, CH)]], wrb_v, sem).wait()

    def add_row(r, _):
      xp_v[r, pl.ds(0, L)] = xp_v[r, pl.ds(0, L)] + xu_v[r, pl.ds(0, L)]
      xp_v[r, pl.ds(L, L)] = xp_v[r, pl.ds(L, L)] + xu_v[r, pl.ds(L, L)]
      return 0

    lax.fori_loop(0, CH, add_row, 0)
    pltpu.sync_copy(xp_v, x_out.at[pl.ds(obase, CH)])
    pltpu.sync_copy(wrv_v, wr_out.at[pl.ds(obase, CH)])
    pltpu.sync_copy(wrb_v, wrb_out.at[pl.ds(obase, CH)])
    return 0

  lax.fori_loop(0, nch, chunk, 0)


def _state_gather_body(B, U, uidx_hbm, dayv_hbm, h_hbm, C_hbm, hpcp_out,
                       seg_v, evlist_v, uvlist_v, u128_v, ev128_v,
                       winh_v, winc_v, st_v, dayv_v, sem):
  tr, last = _tile_range(U)
  wid = _worker_id()
  lo = wid * tr
  pltpu.sync_copy(dayv_hbm, dayv_v)
  pltpu.sync_copy(uidx_hbm, seg_v)
  ds_ = jnp.max(dayv_v[...])
  iot = _iota()

  def do_windows(width, nsw):
    for sw in range(nsw):
      slo = lo + sw * width
      shi = jnp.minimum(slo + width, U)
      wdst_h = winh_v.at[:, pl.ds(0, width)]
      wdst_c = winc_v.at[:, pl.ds(0, width)]
      cp_h = pltpu.async_copy(h_hbm.at[ds_, :, pl.ds(slo, width)], wdst_h,
                              sem)
      cp_c = pltpu.async_copy(C_hbm.at[ds_, :, pl.ds(slo, width)], wdst_c,
                              sem)

      def scan(i, m):
        uv = seg_v[pl.ds(i * L, L)]
        mine = (uv >= slo) & (uv < shi)
        plsc.store_compressed(evlist_v.at[pl.ds(m, L)], i * L + iot,
                              mask=mine)
        plsc.store_compressed(uvlist_v.at[pl.ds(m, L)], uv - slo,
                              mask=mine)
        return m + jnp.max(plsc.all_reduce_population_count(mine))

      m_tot = lax.fori_loop(0, B // L, scan, jnp.int32(0))
      cp_h.wait()
      cp_c.wait()

      @pl.when(m_tot > 0)
      def _():
        nch = (m_tot + CH - 1) // CH

        def chunk(c, _):
          for s in range(CH // L):
            pos = jnp.minimum(c * CH + s * L + iot, m_tot - 1)
            u128_v[pl.ds(s * L, L)] = plsc.load_gather(uvlist_v, [pos])
            ev128_v[pl.ds(s * L, L)] = plsc.load_gather(evlist_v, [pos])

          def ext(e, _):
            ub = plsc.load_gather(u128_v, [jnp.full((L,), e, jnp.int32)])
            st_v[e, pl.ds(0, L)] = plsc.load_gather(winh_v, [iot, ub])
            st_v[e, pl.ds(L, L)] = plsc.load_gather(winh_v, [iot + L, ub])
            st_v[e, pl.ds(2 * L, L)] = plsc.load_gather(winc_v, [iot, ub])
            st_v[e, pl.ds(3 * L, L)] = plsc.load_gather(winc_v,
                                                        [iot + L, ub])
            return 0

          lax.fori_loop(0, CH, ext, 0)
          pltpu.async_copy(st_v, hpcp_out.at[ev128_v], sem).wait()
          return 0

        lax.fori_loop(0, nch, chunk, 0)

  last_pad = (last + 127) // 128 * 128

  @pl.when(wid < NW - 1)
  def _():
    do_windows(tr // 5, 5)

  @pl.when(wid == NW - 1)
  def _():
    do_windows(last_pad, 1)


def _tile_range(U):
  tr = (((U + NW - 1) // NW) + 127) // 128 * 128
  last = U - (NW - 1) * tr
  assert 0 < last <= tr
  return tr, last


def _scatter_body(B, U, uidx_hbm, hncn_hbm, day1v_hbm, h_ref, C_ref,
                  seg_v, w_v, wulist_v, welist_v, u128_v, ev128_v,
                  win_v, winsc_v, hv_v, day1_v, sem):
  tr, last = _tile_range(U)
  wid = _worker_id()
  lo = wid * tr
  hi = jnp.minimum(lo + tr, U)
  pltpu.sync_copy(day1v_hbm, day1_v)
  d1s = jnp.max(day1_v[...])
  iot = _iota()
  neg1 = jnp.full((L,), -1, jnp.int32)

  def winit(j, _):
    w_v[pl.ds(j * L, L)] = neg1
    return 0

  lax.fori_loop(0, tr // L, winit, 0)

  pltpu.sync_copy(uidx_hbm, seg_v)

  def wpass(i, _):
    uv = seg_v[pl.ds(i * L, L)]
    mine = (uv >= lo) & (uv < hi)
    rel = jnp.where(mine, uv - lo, 0)
    plsc.store_scatter(w_v, [rel], i * L + iot, mask=mine)
    return 0

  lax.fori_loop(0, B // L, wpass, 0)

  def do_windows(width, nsw):
    for sw in range(nsw):
      swbase = sw * width
      slo = lo + swbase

      def enum(j, m):
        wv = w_v[pl.ds(swbase + j * L, L)]
        has = wv >= 0
        plsc.store_compressed(wulist_v.at[pl.ds(m, L)], j * L + iot,
                              mask=has)
        plsc.store_compressed(welist_v.at[pl.ds(m, L)], wv, mask=has)
        return m + jnp.max(plsc.all_reduce_population_count(has))

      dsth = win_v.at[:, pl.ds(0, width)]
      dstc = winsc_v.at[:, pl.ds(0, width)]
      cph = pltpu.async_copy(h_ref.at[d1s, :, pl.ds(slo, width)], dsth, sem)
      cpc = pltpu.async_copy(C_ref.at[d1s, :, pl.ds(slo, width)], dstc, sem)
      m_tot = lax.fori_loop(0, width // L, enum, jnp.int32(0))
      cph.wait()
      cpc.wait()

      @pl.when(m_tot > 0)
      def _():
        nch = (m_tot + CH - 1) // CH

        def chunk(c, _):
          for s in range(CH // L):
            pos = jnp.minimum(c * CH + s * L + iot, m_tot - 1)
            u128_v[pl.ds(s * L, L)] = plsc.load_gather(wulist_v, [pos])
            ev128_v[pl.ds(s * L, L)] = plsc.load_gather(welist_v, [pos])
          pltpu.async_copy(hncn_hbm.at[ev128_v], hv_v, sem).wait()

          def ins(e, _):
            ub = plsc.load_gather(u128_v, [jnp.full((L,), e, jnp.int32)])
            plsc.store_scatter(win_v, [iot, ub], hv_v[e, pl.ds(0, L)])
            plsc.store_scatter(win_v, [iot + L, ub], hv_v[e, pl.ds(L, L)])
            plsc.store_scatter(winsc_v, [iot, ub],
                               hv_v[e, pl.ds(H_LANES, L)])
            plsc.store_scatter(winsc_v, [iot + L, ub],
                               hv_v[e, pl.ds(H_LANES + L, L)])
            return 0

          lax.fori_loop(0, CH, ins, 0)
          return 0

        lax.fori_loop(0, nch, chunk, 0)

      cpo = pltpu.async_copy(dsth, h_ref.at[d1s, :, pl.ds(slo, width)], sem)
      cpo2 = pltpu.async_copy(dstc, C_ref.at[d1s, :, pl.ds(slo, width)], sem)
      cpo.wait()
      cpo2.wait()

  last_pad = (last + 127) // 128 * 128

  @pl.when(wid < NW - 1)
  def _():
    do_windows(tr // 5, 5)

  @pl.when(wid == NW - 1)
  def _():
    do_windows(last_pad, 1)


def _copy_body(hi_ref, ci_ref, ho_ref, co_ref):
  ho_ref[...] = hi_ref[...]
  co_ref[...] = ci_ref[...]


def _lstm_body(x_ref, hpcp_ref, wr_ref, wrb_ref, wcat_ref, ucat_ref,
               bcat_ref, hncn_ref, sc_ref):
  x = x_ref[...]
  h = x.shape[1]
  hp = hpcp_ref[:, :h]
  cp = hpcp_ref[:, h:2 * h]
  g = (jnp.dot(x, wcat_ref[...], preferred_element_type=jnp.float32)
       + jnp.dot(hp, ucat_ref[...], preferred_element_type=jnp.float32)
       + bcat_ref[...])
  gi = jax.nn.sigmoid(g[:, :h])
  gf = jax.nn.sigmoid(g[:, h:2 * h])
  go = jax.nn.sigmoid(g[:, 2 * h:3 * h])
  gc = jnp.tanh(g[:, 3 * h:])
  cn = gf * cp + gi * gc
  hn = go * jnp.tanh(cn)
  pad = jnp.zeros((x.shape[0], 2 * h), jnp.float32)
  hncn_ref[...] = jnp.concatenate([hn, cn, pad], axis=1)
  s = jnp.sum(hn * wr_ref[...], axis=1)[None, None, :] + wrb_ref[...]
  sc_ref[...] = jax.nn.sigmoid(s)


def kernel(user_idx, emb_idx, problem_idx, day, h, C, user_emb, problem_emb,
           Wi, Wf, Wo, Wc, Ui, Uf, Uo, Uc, Ui_b, Uf_b, Uo_b, Uc_b, Wr, Wr_b):
  B = user_idx.shape[0]
  U, H, D1 = h.shape
  uidx = user_idx.astype(jnp.int32)
  eidx = emb_idx.astype(jnp.int32)
  pidx = problem_idx.astype(jnp.int32)
  dayv = jnp.full((L,), day, jnp.int32)
  mesh = plsc.VectorSubcoreMesh(core_axis_name="c", subcore_axis_name="s",
                                num_cores=NC, num_subcores=NS)
  sc_params = pltpu.CompilerParams(needs_layout_passes=False,
                                   use_tc_tiling_on_sc=False)

  f32 = jnp.float32
  i32 = jnp.int32
  ept = B // NW

  gather = pl.kernel(
      functools.partial(_gather_body, B, H),
      out_type=(
          jax.ShapeDtypeStruct((B, H), f32),
          jax.ShapeDtypeStruct((B, H), f32),
          jax.ShapeDtypeStruct((B,), f32),
      ),
      mesh=mesh,
      scratch_types=[
          pltpu.VMEM((ept,), i32),
          pltpu.VMEM((ept,), i32),
          pltpu.VMEM((ept,), i32),
          pltpu.VMEM((CH, H), f32),
          pltpu.VMEM((CH, H), f32),
          pltpu.VMEM((CH, H), f32),
          pltpu.VMEM((CH,), f32),
          pltpu.SemaphoreType.DMA,
      ],
      compiler_params=sc_params,
      name="sc_lstm_gather",
  )
  x, wr, wrb = gather(uidx, eidx, pidx, problem_emb, user_emb, Wr, Wr_b)

  tr, _last = _tile_range(U)
  seg = 4096
  wmax = max(tr // 5, (_last + 127) // 128 * 128)
  hT = jnp.transpose(h, (2, 1, 0))
  CT = jnp.transpose(C, (2, 1, 0))
  sc_tiled_params = pltpu.CompilerParams(needs_layout_passes=False,
                                         use_tc_tiling_on_sc=True,
                                         disable_bounds_checks=True)
  state_gather = pl.kernel(
      functools.partial(_state_gather_body, B, U),
      out_type=jax.ShapeDtypeStruct((B, 4 * H), f32),
      mesh=mesh,
      scratch_types=[
          pltpu.VMEM((B,), i32),
          pltpu.VMEM((B + L,), i32),
          pltpu.VMEM((B + L,), i32),
          pltpu.VMEM((CH,), i32),
          pltpu.VMEM((CH,), i32),
          pltpu.VMEM((H, wmax), f32),
          pltpu.VMEM((H, wmax), f32),
          pltpu.VMEM((CH, 4 * H), f32),
          pltpu.VMEM((L,), i32),
          pltpu.SemaphoreType.DMA,
      ],
      compiler_params=sc_tiled_params,
      name="sc_lstm_state_gather",
  )
  hpcp = state_gather(uidx, dayv, hT, CT)

  wcat = jnp.concatenate([Wi.T, Wf.T, Wo.T, Wc.T], axis=1)
  ucat = jnp.concatenate([Ui.T, Uf.T, Uo.T, Uc.T], axis=1)
  bcat = jnp.concatenate([Ui_b, Uf_b, Uo_b, Uc_b])[None, :]

  blk = 512
  nblk = B // blk
  hncn, score2d = pl.pallas_call(
      _lstm_body,
      grid=(nblk,),
      in_specs=[
          pl.BlockSpec((blk, H), lambda i: (i, 0)),
          pl.BlockSpec((blk, 4 * H), lambda i: (i, 0)),
          pl.BlockSpec((blk, H), lambda i: (i, 0)),
          pl.BlockSpec((1, 1, blk), lambda i: (i, 0, 0)),
          pl.BlockSpec((H, 4 * H), lambda i: (0, 0)),
          pl.BlockSpec((H, 4 * H), lambda i: (0, 0)),
          pl.BlockSpec((1, 4 * H), lambda i: (0, 0)),
      ],
      out_specs=[
          pl.BlockSpec((blk, 4 * H), lambda i: (i, 0)),
          pl.BlockSpec((1, 1, blk), lambda i: (i, 0, 0)),
      ],
      out_shape=[
          jax.ShapeDtypeStruct((B, 4 * H), f32),
          jax.ShapeDtypeStruct((nblk, 1, blk), f32),
      ],
      name="tc_lstm_gates",
  )(x, hpcp, wr, wrb.reshape(nblk, 1, blk), wcat, ucat, bcat)
  score = score2d.reshape(B)

  lcap = wmax + L
  cb = 50048
  ncb = (U + cb - 1) // cb
  hTc, CTc = pl.pallas_call(
      _copy_body,
      grid=(D1, ncb),
      in_specs=[
          pl.BlockSpec((1, H, cb), lambda i, j: (i, 0, j)),
          pl.BlockSpec((1, H, cb), lambda i, j: (i, 0, j)),
      ],
      out_specs=[
          pl.BlockSpec((1, H, cb), lambda i, j: (i, 0, j)),
          pl.BlockSpec((1, H, cb), lambda i, j: (i, 0, j)),
      ],
      out_shape=[
          jax.ShapeDtypeStruct((D1, H, U), f32),
          jax.ShapeDtypeStruct((D1, H, U), f32),
      ],
      name="tc_state_copy",
  )(hT, CT)
  h_refT = jax.new_ref(hTc)
  C_refT = jax.new_ref(CTc)
  scatter = pl.kernel(
      functools.partial(_scatter_body, B, U),
      out_type=(),
      mesh=mesh,
      scratch_types=[
          pltpu.VMEM((B,), i32),
          pltpu.VMEM((tr,), i32),
          pltpu.VMEM((lcap,), i32),
          pltpu.VMEM((lcap,), i32),
          pltpu.VMEM((CH,), i32),
          pltpu.VMEM((CH,), i32),
          pltpu.VMEM((H, wmax), f32),
          pltpu.VMEM((H, wmax), f32),
          pltpu.VMEM((CH, 4 * H), f32),
          pltpu.VMEM((L,), i32),
          pltpu.SemaphoreType.DMA,
      ],
      compiler_params=pltpu.CompilerParams(needs_layout_passes=False,
                                           use_tc_tiling_on_sc=True,
                                           disable_bounds_checks=True),
      name="sc_lstm_scatter",
  )
  scatter(uidx, hncn, dayv + 1, h_refT, C_refT)
  h_out = jnp.transpose(jax.freeze(h_refT), (2, 1, 0))
  C_out = jnp.transpose(jax.freeze(C_refT), (2, 1, 0))
  return h_out, C_out, score


# speedup vs baseline: 11.3805x; 1.0015x over previous
import functools

import jax
import jax.numpy as jnp
from jax import lax
from jax.experimental import pallas as pl
from jax.experimental.pallas import tpu as pltpu
from jax.experimental.pallas import tpu_sc as plsc

NC = 2
NS = 16
NW = NC * NS
L = 16
CH = 128
H_LANES = 32


def _worker_id():
  return lax.axis_index("s") * NC + lax.axis_index("c")


def _iota():
  return lax.iota(jnp.int32, L)


def _gather_body(B, H, uidx_hbm, eidx_hbm, pidx_hbm, pemb_hbm, uemb_hbm,
                 wr_hbm, wrb_hbm, x_out, wr_out, wrb_out,
                 uidx_v, eidx_v, pidx_v, xp_v, xu_v, wrv_v, wrb_v, sem):
  ept = B // NW
  nch = ept // CH
  wid = _worker_id()
  base = wid * ept
  pltpu.sync_copy(uidx_hbm.at[pl.ds(base, ept)], uidx_v)
  pltpu.sync_copy(eidx_hbm.at[pl.ds(base, ept)], eidx_v)
  pltpu.sync_copy(pidx_hbm.at[pl.ds(base, ept)], pidx_v)

  def chunk(c, _):
    off = c * CH
    obase = base + off
    cp1 = pltpu.async_copy(pemb_hbm.at[eidx_v.at[pl.ds(off, CH)]], xp_v, sem)
    cp2 = pltpu.async_copy(uemb_hbm.at[uidx_v.at[pl.ds(off, CH)]], xu_v, sem)
    cp3 = pltpu.async_copy(wr_hbm.at[pidx_v.at[pl.ds(off, CH)]], wrv_v, sem)
    cp4 = pltpu.async_copy(wrb_hbm.at[pidx_v.at[pl.ds(off, CH)]], wrb_v, sem)
    cp1.wait()
    cp2.wait()
    cp3.wait()
    cp4.wait()

    def add_row(r, _):
      xp_v[r, pl.ds(0, L)] = xp_v[r, pl.ds(0, L)] + xu_v[r, pl.ds(0, L)]
      xp_v[r, pl.ds(L, L)] = xp_v[r, pl.ds(L, L)] + xu_v[r, pl.ds(L, L)]
      return 0

    lax.fori_loop(0, CH, add_row, 0)
    pltpu.sync_copy(xp_v, x_out.at[pl.ds(obase, CH)])
    pltpu.sync_copy(wrv_v, wr_out.at[pl.ds(obase, CH)])
    pltpu.sync_copy(wrb_v, wrb_out.at[pl.ds(obase, CH)])
    return 0

  lax.fori_loop(0, nch, chunk, 0)


def _state_gather_body(B, U, uidx_hbm, dayv_hbm, h_hbm, C_hbm, hpcp_out,
                       seg_v, evlist_v, uvlist_v, u128_v, ev128_v,
                       winh_v, winc_v, st_v, dayv_v, sem):
  tr, last = _tile_range(U)
  wid = _worker_id()
  lo = wid * tr
  pltpu.sync_copy(dayv_hbm, dayv_v)
  pltpu.sync_copy(uidx_hbm, seg_v)
  ds_ = jnp.max(dayv_v[...])
  iot = _iota()

  def do_windows(width, nsw):
    for sw in range(nsw):
      slo = lo + sw * width
      shi = jnp.minimum(slo + width, U)
      wdst_h = winh_v.at[:, pl.ds(0, width)]
      wdst_c = winc_v.at[:, pl.ds(0, width)]
      cp_h = pltpu.async_copy(h_hbm.at[ds_, :, pl.ds(slo, width)], wdst_h,
                              sem)
      cp_c = pltpu.async_copy(C_hbm.at[ds_, :, pl.ds(slo, width)], wdst_c,
                              sem)

      def scan(i, m):
        uv = seg_v[pl.ds(i * L, L)]
        mine = (uv >= slo) & (uv < shi)
        plsc.store_compressed(evlist_v.at[pl.ds(m, L)], i * L + iot,
                              mask=mine)
        plsc.store_compressed(uvlist_v.at[pl.ds(m, L)], uv - slo,
                              mask=mine)
        return m + jnp.max(plsc.all_reduce_population_count(mine))

      m_tot = lax.fori_loop(0, B // L, scan, jnp.int32(0))
      cp_h.wait()
      cp_c.wait()

      @pl.when(m_tot > 0)
      def _():
        nch = (m_tot + CH - 1) // CH

        def chunk(c, _):
          for s in range(CH // L):
            pos = jnp.minimum(c * CH + s * L + iot, m_tot - 1)
            u128_v[pl.ds(s * L, L)] = plsc.load_gather(uvlist_v, [pos])
            ev128_v[pl.ds(s * L, L)] = plsc.load_gather(evlist_v, [pos])

          def ext(e, _):
            ub = plsc.load_gather(u128_v, [jnp.full((L,), e, jnp.int32)])
            st_v[e, pl.ds(0, L)] = plsc.load_gather(winh_v, [iot, ub])
            st_v[e, pl.ds(L, L)] = plsc.load_gather(winh_v, [iot + L, ub])
            st_v[e, pl.ds(2 * L, L)] = plsc.load_gather(winc_v, [iot, ub])
            st_v[e, pl.ds(3 * L, L)] = plsc.load_gather(winc_v,
                                                        [iot + L, ub])
            return 0

          lax.fori_loop(0, CH, ext, 0)
          pltpu.async_copy(st_v, hpcp_out.at[ev128_v], sem).wait()
          return 0

        lax.fori_loop(0, nch, chunk, 0)

  last_pad = (last + 127) // 128 * 128

  @pl.when(wid < NW - 1)
  def _():
    do_windows(tr // 5, 5)

  @pl.when(wid == NW - 1)
  def _():
    do_windows(last_pad, 1)


def _tile_range(U):
  tr = (((U + NW - 1) // NW) + 127) // 128 * 128
  last = U - (NW - 1) * tr
  assert 0 < last <= tr
  return tr, last


def _scatter_body(B, U, uidx_hbm, hncn_hbm, day1v_hbm, h_ref, C_ref,
                  seg_v, w_v, wulist_v, welist_v, u128_v, ev128_v,
                  win_v, winsc_v, hv_v, day1_v, sem):
  tr, last = _tile_range(U)
  wid = _worker_id()
  lo = wid * tr
  hi = jnp.minimum(lo + tr, U)
  pltpu.sync_copy(day1v_hbm, day1_v)
  d1s = jnp.max(day1_v[...])
  iot = _iota()
  neg1 = jnp.full((L,), -1, jnp.int32)

  def winit(j, _):
    w_v[pl.ds(j * L, L)] = neg1
    return 0

  lax.fori_loop(0, tr // L, winit, 0)

  pltpu.sync_copy(uidx_hbm, seg_v)

  def wpass(i, _):
    uv = seg_v[pl.ds(i * L, L)]
    mine = (uv >= lo) & (uv < hi)
    rel = jnp.where(mine, uv - lo, 0)
    plsc.store_scatter(w_v, [rel], i * L + iot, mask=mine)
    return 0

  lax.fori_loop(0, B // L, wpass, 0)

  def do_windows(width, nsw):
    for sw in range(nsw):
      swbase = sw * width
      slo = lo + swbase

      def enum(j, m):
        wv = w_v[pl.ds(swbase + j * L, L)]
        has = wv >= 0
        plsc.store_compressed(wulist_v.at[pl.ds(m, L)], j * L + iot,
                              mask=has)
        plsc.store_compressed(welist_v.at[pl.ds(m, L)], wv, mask=has)
        return m + jnp.max(plsc.all_reduce_population_count(has))

      dsth = win_v.at[:, pl.ds(0, width)]
      dstc = winsc_v.at[:, pl.ds(0, width)]
      cph = pltpu.async_copy(h_ref.at[d1s, :, pl.ds(slo, width)], dsth, sem)
      cpc = pltpu.async_copy(C_ref.at[d1s, :, pl.ds(slo, width)], dstc, sem)
      m_tot = lax.fori_loop(0, width // L, enum, jnp.int32(0))
      cph.wait()
      cpc.wait()

      @pl.when(m_tot > 0)
      def _():
        nch = (m_tot + CH - 1) // CH

        def chunk(c, _):
          for s in range(CH // L):
            pos = jnp.minimum(c * CH + s * L + iot, m_tot - 1)
            u128_v[pl.ds(s * L, L)] = plsc.load_gather(wulist_v, [pos])
            ev128_v[pl.ds(s * L, L)] = plsc.load_gather(welist_v, [pos])
          pltpu.async_copy(hncn_hbm.at[ev128_v], hv_v, sem).wait()

          def ins(e, _):
            ub = plsc.load_gather(u128_v, [jnp.full((L,), e, jnp.int32)])
            plsc.store_scatter(win_v, [iot, ub], hv_v[e, pl.ds(0, L)])
            plsc.store_scatter(win_v, [iot + L, ub], hv_v[e, pl.ds(L, L)])
            plsc.store_scatter(winsc_v, [iot, ub],
                               hv_v[e, pl.ds(H_LANES, L)])
            plsc.store_scatter(winsc_v, [iot + L, ub],
                               hv_v[e, pl.ds(H_LANES + L, L)])
            return 0

          lax.fori_loop(0, CH, ins, 0)
          return 0

        lax.fori_loop(0, nch, chunk, 0)

      cpo = pltpu.async_copy(dsth, h_ref.at[d1s, :, pl.ds(slo, width)], sem)
      cpo2 = pltpu.async_copy(dstc, C_ref.at[d1s, :, pl.ds(slo, width)], sem)
      cpo.wait()
      cpo2.wait()

  last_pad = (last + 127) // 128 * 128

  @pl.when(wid < NW - 1)
  def _():
    do_windows(tr // 5, 5)

  @pl.when(wid == NW - 1)
  def _():
    do_windows(last_pad, 1)


def _copy_body(hi_ref, ci_ref, ho_ref, co_ref):
  ho_ref[...] = hi_ref[...]
  co_ref[...] = ci_ref[...]


def _lstm_body(x_ref, hpcp_ref, wr_ref, wrb_ref, wcat_ref, ucat_ref,
               bcat_ref, hncn_ref, sc_ref):
  x = x_ref[...]
  h = x.shape[1]
  hp = hpcp_ref[:, :h]
  cp = hpcp_ref[:, h:2 * h]
  g = (jnp.dot(x, wcat_ref[...], preferred_element_type=jnp.float32)
       + jnp.dot(hp, ucat_ref[...], preferred_element_type=jnp.float32)
       + bcat_ref[...])
  gi = jax.nn.sigmoid(g[:, :h])
  gf = jax.nn.sigmoid(g[:, h:2 * h])
  go = jax.nn.sigmoid(g[:, 2 * h:3 * h])
  gc = jnp.tanh(g[:, 3 * h:])
  cn = gf * cp + gi * gc
  hn = go * jnp.tanh(cn)
  pad = jnp.zeros((x.shape[0], 2 * h), jnp.float32)
  hncn_ref[...] = jnp.concatenate([hn, cn, pad], axis=1)
  s = jnp.sum(hn * wr_ref[...], axis=1)[None, None, :] + wrb_ref[...]
  sc_ref[...] = jax.nn.sigmoid(s)


def kernel(user_idx, emb_idx, problem_idx, day, h, C, user_emb, problem_emb,
           Wi, Wf, Wo, Wc, Ui, Uf, Uo, Uc, Ui_b, Uf_b, Uo_b, Uc_b, Wr, Wr_b):
  B = user_idx.shape[0]
  U, H, D1 = h.shape
  uidx = user_idx.astype(jnp.int32)
  eidx = emb_idx.astype(jnp.int32)
  pidx = problem_idx.astype(jnp.int32)
  dayv = jnp.full((L,), day, jnp.int32)
  mesh = plsc.VectorSubcoreMesh(core_axis_name="c", subcore_axis_name="s",
                                num_cores=NC, num_subcores=NS)
  sc_params = pltpu.CompilerParams(needs_layout_passes=False,
                                   use_tc_tiling_on_sc=False)

  f32 = jnp.float32
  i32 = jnp.int32
  ept = B // NW

  gather = pl.kernel(
      functools.partial(_gather_body, B, H),
      out_type=(
          jax.ShapeDtypeStruct((B, H), f32),
          jax.ShapeDtypeStruct((B, H), f32),
          jax.ShapeDtypeStruct((B,), f32),
      ),
      mesh=mesh,
      scratch_types=[
          pltpu.VMEM((ept,), i32),
          pltpu.VMEM((ept,), i32),
          pltpu.VMEM((ept,), i32),
          pltpu.VMEM((CH, H), f32),
          pltpu.VMEM((CH, H), f32),
          pltpu.VMEM((CH, H), f32),
          pltpu.VMEM((CH,), f32),
          pltpu.SemaphoreType.DMA,
      ],
      compiler_params=sc_params,
      name="sc_lstm_gather",
  )
  x, wr, wrb = gather(uidx, eidx, pidx, problem_emb, user_emb, Wr, Wr_b)

  tr, _last = _tile_range(U)
  seg = 4096
  wmax = max(tr // 5, (_last + 127) // 128 * 128)
  hT = jnp.transpose(h, (2, 1, 0))
  CT = jnp.transpose(C, (2, 1, 0))
  sc_tiled_params = pltpu.CompilerParams(needs_layout_passes=False,
                                         use_tc_tiling_on_sc=True,
                                         disable_bounds_checks=True)
  state_gather = pl.kernel(
      functools.partial(_state_gather_body, B, U),
      out_type=jax.ShapeDtypeStruct((B, 4 * H), f32),
      mesh=mesh,
      scratch_types=[
          pltpu.VMEM((B,), i32),
          pltpu.VMEM((B + L,), i32),
          pltpu.VMEM((B + L,), i32),
          pltpu.VMEM((CH,), i32),
          pltpu.VMEM((CH,), i32),
          pltpu.VMEM((H, wmax), f32),
          pltpu.VMEM((H, wmax), f32),
          pltpu.VMEM((CH, 4 * H), f32),
          pltpu.VMEM((L,), i32),
          pltpu.SemaphoreType.DMA,
      ],
      compiler_params=sc_tiled_params,
      name="sc_lstm_state_gather",
  )
  hpcp = state_gather(uidx, dayv, hT, CT)

  wcat = jnp.concatenate([Wi.T, Wf.T, Wo.T, Wc.T], axis=1)
  ucat = jnp.concatenate([Ui.T, Uf.T, Uo.T, Uc.T], axis=1)
  bcat = jnp.concatenate([Ui_b, Uf_b, Uo_b, Uc_b])[None, :]

  blk = 512
  nblk = B // blk
  hncn, score2d = pl.pallas_call(
      _lstm_body,
      grid=(nblk,),
      in_specs=[
          pl.BlockSpec((blk, H), lambda i: (i, 0)),
          pl.BlockSpec((blk, 4 * H), lambda i: (i, 0)),
          pl.BlockSpec((blk, H), lambda i: (i, 0)),
          pl.BlockSpec((1, 1, blk), lambda i: (i, 0, 0)),
          pl.BlockSpec((H, 4 * H), lambda i: (0, 0)),
          pl.BlockSpec((H, 4 * H), lambda i: (0, 0)),
          pl.BlockSpec((1, 4 * H), lambda i: (0, 0)),
      ],
      out_specs=[
          pl.BlockSpec((blk, 4 * H), lambda i: (i, 0)),
          pl.BlockSpec((1, 1, blk), lambda i: (i, 0, 0)),
      ],
      out_shape=[
          jax.ShapeDtypeStruct((B, 4 * H), f32),
          jax.ShapeDtypeStruct((nblk, 1, blk), f32),
      ],
      name="tc_lstm_gates",
  )(x, hpcp, wr, wrb.reshape(nblk, 1, blk), wcat, ucat, bcat)
  score = score2d.reshape(B)

  lcap = wmax + L
  cb = 50048
  ncb = (U + cb - 1) // cb
  hTc, CTc = pl.pallas_call(
      _copy_body,
      grid=(D1, ncb),
      in_specs=[
          pl.BlockSpec((1, H, cb), lambda i, j: (i, 0, j)),
          pl.BlockSpec((1, H, cb), lambda i, j: (i, 0, j)),
      ],
      out_specs=[
          pl.BlockSpec((1, H, cb), lambda i, j: (i, 0, j)),
          pl.BlockSpec((1, H, cb), lambda i, j: (i, 0, j)),
      ],
      out_shape=[
          jax.ShapeDtypeStruct((D1, H, U), f32),
          jax.ShapeDtypeStruct((D1, H, U), f32),
      ],
      name="tc_state_copy",
  )(hT, CT)
  h_refT = jax.new_ref(hTc)
  C_refT = jax.new_ref(CTc)
  scatter = pl.kernel(
      functools.partial(_scatter_body, B, U),
      out_type=(),
      mesh=mesh,
      scratch_types=[
          pltpu.VMEM((B,), i32),
          pltpu.VMEM((tr,), i32),
          pltpu.VMEM((lcap,), i32),
          pltpu.VMEM((lcap,), i32),
          pltpu.VMEM((CH,), i32),
          pltpu.VMEM((CH,), i32),
          pltpu.VMEM((H, wmax), f32),
          pltpu.VMEM((H, wmax), f32),
          pltpu.VMEM((CH, 4 * H), f32),
          pltpu.VMEM((L,), i32),
          pltpu.SemaphoreType.DMA,
      ],
      compiler_params=pltpu.CompilerParams(needs_layout_passes=False,
                                           use_tc_tiling_on_sc=True,
                                           disable_bounds_checks=True),
      name="sc_lstm_scatter",
  )
  scatter(uidx, hncn, dayv + 1, h_refT, C_refT)
  h_out = jnp.transpose(jax.freeze(h_refT), (2, 1, 0))
  C_out = jnp.transpose(jax.freeze(C_refT), (2, 1, 0))
  return h_out, C_out, score
